# Initial kernel scaffold; baseline (speedup 1.0000x reference)
import jax
import jax.numpy as jnp
from jax.experimental import pallas as pl


def kernel(x, pe_w, pe_b, pd_w, down_w, up_w, ce_w, ce_b, cd_w, dec_bias):
    raise NotImplementedError("write your pallas kernel here")


# speedup vs baseline: 7.4436x; 7.4436x over previous
import jax
import jax.numpy as jnp
from jax.experimental import pallas as pl
from jax.experimental.pallas import tpu as pltpu

INPUT_DIM = 768
N_PARENTS = 64
SUBSPACE = 32
N_CHILD = 128
BT = 256

_NEG = -3.0e38


def _top2_codes(logits, n):
    iota = jax.lax.broadcasted_iota(jnp.int32, logits.shape, 1)
    m1 = jnp.max(logits, axis=1, keepdims=True)
    i1 = jnp.min(jnp.where(logits == m1, iota, n), axis=1, keepdims=True)
    sel1 = iota == i1
    rest = jnp.where(sel1, _NEG, logits)
    m2 = jnp.max(rest, axis=1, keepdims=True)
    i2 = jnp.min(jnp.where(rest == m2, iota, n), axis=1, keepdims=True)
    return jnp.logical_or(sel1, iota == i2)


def _body(x_ref, pe_wt_ref, pe_b_ref, pd_w_ref, down_ref, up_ref,
          ce_t_ref, ce_b_ref, cd_t_ref, bias_ref, out_ref, xsub_ref, acc_ref):
    xb = x_ref[:]
    p_logits = jnp.dot(xb, pe_wt_ref[:],
                       preferred_element_type=jnp.float32) + pe_b_ref[:]
    pcodes = _top2_codes(p_logits, N_PARENTS).astype(jnp.float32)
    recon = jnp.dot(pcodes, pd_w_ref[:], preferred_element_type=jnp.float32)
    xsub_ref[:] = jnp.dot(xb, down_ref[:],
                          preferred_element_type=jnp.float32)
    for p in range(N_PARENTS):
        xs = xsub_ref[:, p * SUBSPACE:(p + 1) * SUBSPACE]
        cl = jnp.dot(xs, ce_t_ref[p],
                     preferred_element_type=jnp.float32) + ce_b_ref[p:p + 1, :]
        ccodes = _top2_codes(cl, N_CHILD).astype(jnp.float32)
        csub = jnp.dot(ccodes, cd_t_ref[p],
                       preferred_element_type=jnp.float32)
        acc_ref[:, p * SUBSPACE:(p + 1) * SUBSPACE] = csub * pcodes[:, p:p + 1]
    out_ref[:] = (recon
                  + jnp.dot(acc_ref[:], up_ref[:],
                            preferred_element_type=jnp.float32)
                  + bias_ref[:])


def kernel(x, pe_w, pe_b, pd_w, down_w, up_w, ce_w, ce_b, cd_w, dec_bias):
    tokens = x.shape[0]
    assert tokens % BT == 0
    pe_wt = pe_w.T
    down_all = jnp.transpose(down_w, (2, 0, 1)).reshape(INPUT_DIM,
                                                        N_PARENTS * SUBSPACE)
    up_all = jnp.transpose(up_w, (0, 2, 1)).reshape(N_PARENTS * SUBSPACE,
                                                    INPUT_DIM)
    ce_t = jnp.transpose(ce_w, (0, 2, 1))
    cd_t = jnp.transpose(cd_w, (0, 2, 1))
    pe_b2 = pe_b.reshape(1, N_PARENTS)
    bias2 = dec_bias.reshape(1, INPUT_DIM)

    const = lambda *dims: pl.BlockSpec(dims, lambda i: (0,) * len(dims))
    return pl.pallas_call(
        _body,
        grid=(tokens // BT,),
        in_specs=[
            pl.BlockSpec((BT, INPUT_DIM), lambda i: (i, 0)),
            const(INPUT_DIM, N_PARENTS),
            const(1, N_PARENTS),
            const(N_PARENTS, INPUT_DIM),
            const(INPUT_DIM, N_PARENTS * SUBSPACE),
            const(N_PARENTS * SUBSPACE, INPUT_DIM),
            const(N_PARENTS, SUBSPACE, N_CHILD),
            const(N_PARENTS, N_CHILD),
            const(N_PARENTS, N_CHILD, SUBSPACE),
            const(1, INPUT_DIM),
        ],
        out_specs=pl.BlockSpec((BT, INPUT_DIM), lambda i: (i, 0)),
        out_shape=jax.ShapeDtypeStruct((tokens, INPUT_DIM), jnp.float32),
        scratch_shapes=[
            pltpu.VMEM((BT, N_PARENTS * SUBSPACE), jnp.float32),
            pltpu.VMEM((BT, N_PARENTS * SUBSPACE), jnp.float32),
        ],
    )(x, pe_wt, pe_b2, pd_w, down_all, up_all, ce_t, ce_b, cd_t, bias2)


# speedup vs baseline: 10.0405x; 1.3489x over previous
import jax
import jax.numpy as jnp
from jax.experimental import pallas as pl
from jax.experimental.pallas import tpu as pltpu

INPUT_DIM = 768
N_PARENTS = 64
SUBSPACE = 32
N_CHILD = 128
BT = 256

_NEG = -3.0e38


def _top2_codes(logits, iota_f):
    big = 1.0e9
    m1 = jnp.max(logits, axis=1, keepdims=True)
    i1 = jnp.min(jnp.where(logits == m1, iota_f, big), axis=1, keepdims=True)
    sel1 = iota_f == i1
    rest = jnp.where(sel1, _NEG, logits)
    m2 = jnp.max(rest, axis=1, keepdims=True)
    i2 = jnp.min(jnp.where(rest == m2, iota_f, big), axis=1, keepdims=True)
    return jnp.logical_or(sel1, iota_f == i2)


def _body(x_ref, pe_wt_ref, pe_b_ref, pd_w_ref, down_ref, up_ref,
          ce_t_ref, ce_b_ref, cd_t_ref, bias_ref, out_ref, xsub_ref, acc_ref):
    xb = x_ref[:]
    iota_p = jax.lax.broadcasted_iota(
        jnp.int32, (BT, N_PARENTS), 1).astype(jnp.float32)
    iota_c = jax.lax.broadcasted_iota(
        jnp.int32, (BT, N_CHILD), 1).astype(jnp.float32)
    p_logits = jnp.dot(xb, pe_wt_ref[:],
                       preferred_element_type=jnp.float32) + pe_b_ref[:]
    pcodes = jnp.where(_top2_codes(p_logits, iota_p), 1.0, 0.0)
    recon = jnp.dot(pcodes, pd_w_ref[:], preferred_element_type=jnp.float32)
    xsub_ref[:] = jnp.dot(xb, down_ref[:],
                          preferred_element_type=jnp.float32)
    for p in range(N_PARENTS):
        xs = xsub_ref[:, p * SUBSPACE:(p + 1) * SUBSPACE]
        cl = jnp.dot(xs, ce_t_ref[p],
                     preferred_element_type=jnp.float32) + ce_b_ref[p:p + 1, :]
        ccodes = jnp.where(_top2_codes(cl, iota_c), 1.0, 0.0)
        csub = jnp.dot(ccodes, cd_t_ref[p],
                       preferred_element_type=jnp.float32)
        acc_ref[:, p * SUBSPACE:(p + 1) * SUBSPACE] = csub * pcodes[:, p:p + 1]
    out_ref[:] = (recon
                  + jnp.dot(acc_ref[:], up_ref[:],
                            preferred_element_type=jnp.float32)
                  + bias_ref[:])


def kernel(x, pe_w, pe_b, pd_w, down_w, up_w, ce_w, ce_b, cd_w, dec_bias):
    tokens = x.shape[0]
    assert tokens % BT == 0
    pe_wt = pe_w.T
    down_all = jnp.transpose(down_w, (2, 0, 1)).reshape(INPUT_DIM,
                                                        N_PARENTS * SUBSPACE)
    up_all = jnp.transpose(up_w, (0, 2, 1)).reshape(N_PARENTS * SUBSPACE,
                                                    INPUT_DIM)
    ce_t = jnp.transpose(ce_w, (0, 2, 1))
    cd_t = jnp.transpose(cd_w, (0, 2, 1))
    pe_b2 = pe_b.reshape(1, N_PARENTS)
    bias2 = dec_bias.reshape(1, INPUT_DIM)

    const = lambda *dims: pl.BlockSpec(dims, lambda i: (0,) * len(dims))
    return pl.pallas_call(
        _body,
        grid=(tokens // BT,),
        in_specs=[
            pl.BlockSpec((BT, INPUT_DIM), lambda i: (i, 0)),
            const(INPUT_DIM, N_PARENTS),
            const(1, N_PARENTS),
            const(N_PARENTS, INPUT_DIM),
            const(INPUT_DIM, N_PARENTS * SUBSPACE),
            const(N_PARENTS * SUBSPACE, INPUT_DIM),
            const(N_PARENTS, SUBSPACE, N_CHILD),
            const(N_PARENTS, N_CHILD),
            const(N_PARENTS, N_CHILD, SUBSPACE),
            const(1, INPUT_DIM),
        ],
        out_specs=pl.BlockSpec((BT, INPUT_DIM), lambda i: (i, 0)),
        out_shape=jax.ShapeDtypeStruct((tokens, INPUT_DIM), jnp.float32),
        scratch_shapes=[
            pltpu.VMEM((BT, N_PARENTS * SUBSPACE), jnp.float32),
            pltpu.VMEM((BT, N_PARENTS * SUBSPACE), jnp.float32),
        ],
    )(x, pe_wt, pe_b2, pd_w, down_all, up_all, ce_t, ce_b, cd_t, bias2)


# speedup vs baseline: 22.0617x; 2.1973x over previous
import functools

import jax
import jax.numpy as jnp
from jax import lax
from jax.experimental import pallas as pl
from jax.experimental.pallas import tpu as pltpu
from jax.experimental.pallas import tpu_sc as plsc

D = 768
P = 64
S = 32
C = 128
T = 4096
BLK = 128
NBLOCKS = 128
NROWS = NBLOCKS * BLK
BT1 = 512

_NEG = -3.0e38
_F32 = jnp.float32


def _top2(logits, iota_f):
    big = 1.0e9
    m1 = jnp.max(logits, axis=1, keepdims=True)
    i1 = jnp.min(jnp.where(logits == m1, iota_f, big), axis=1, keepdims=True)
    sel1 = iota_f == i1
    rest = jnp.where(sel1, _NEG, logits)
    m2 = jnp.max(rest, axis=1, keepdims=True)
    i2 = jnp.min(jnp.where(rest == m2, iota_f, big), axis=1, keepdims=True)
    return jnp.logical_or(sel1, iota_f == i2), i1, i2


def _k1_body(x_ref, pewt_ref, peb_ref, pdw_ref, bias_ref, base_ref, pp_ref):
    xb = x_ref[:]
    iota_p = jax.lax.broadcasted_iota(jnp.int32, (BT1, P), 1).astype(_F32)
    logits = jnp.dot(xb, pewt_ref[:], preferred_element_type=_F32) + peb_ref[:]
    mask, i1, i2 = _top2(logits, iota_p)
    pcodes = jnp.where(mask, 1.0, 0.0)
    base_ref[:] = jnp.dot(pcodes, pdw_ref[:],
                          preferred_element_type=_F32) + bias_ref[:]
    packed = i1 * 64.0 + i2
    pp_ref[:] = jnp.broadcast_to(packed, (BT1, 128)).astype(jnp.int32)


def _k1(x, pe_w, pe_b, pd_w, dec_bias):
    const = lambda *dims: pl.BlockSpec(dims, lambda i: (0,) * len(dims))
    return pl.pallas_call(
        _k1_body,
        grid=(T // BT1,),
        in_specs=[
            pl.BlockSpec((BT1, D), lambda i: (i, 0)),
            const(D, P),
            const(1, P),
            const(P, D),
            const(1, D),
        ],
        out_specs=[
            pl.BlockSpec((BT1, D), lambda i: (i, 0)),
            pl.BlockSpec((BT1, 128), lambda i: (i, 0)),
        ],
        out_shape=[
            jax.ShapeDtypeStruct((T, D), _F32),
            jax.ShapeDtypeStruct((T, 128), jnp.int32),
        ],
    )(x, pe_w.T, pe_b.reshape(1, P), pd_w, dec_bias.reshape(1, D))


def _k2a_body(pp_hbm, inv0_hbm, inv1_hbm, idx_hbm, bp_hbm, xch_hbm,
              ppv, hist_pub, histv, inv0v, inv1v, idxvf, bpv,
              runcnt, la, tots, pres, offs, blk0, nbv):
    sid = lax.axis_index("s")
    cid = lax.axis_index("c")
    tpt = T // 16
    lane = lax.iota(jnp.int32, 16)
    pltpu.sync_copy(pp_hbm.at[pl.ds(sid * tpt, tpt)], ppv)

    def zbody(p, _):
        runcnt[p] = 0
        return _

    lax.fori_loop(0, P, zbody, 0)

    def pass1(cc, _):
        for j in range(16):
            i = cc * 16 + j
            pk = ppv[i, pl.ds(0, 16)][0]
            p1 = lax.shift_right_logical(pk, 6)
            p2 = jnp.bitwise_and(pk, 63)
            r1 = runcnt[p1]
            la[2 * i] = p1 * 4096 + r1
            runcnt[p1] = r1 + 1
            r2 = runcnt[p2]
            la[2 * i + 1] = p2 * 4096 + r2
            runcnt[p2] = r2 + 1
        return _

    lax.fori_loop(0, 16, pass1, 0)

    for j in range(P // 16):
        acc = jnp.zeros((16,), jnp.int32)
        for l in range(16):
            acc = jnp.where(lane == l, runcnt[16 * j + l], acc)
        hist_pub[pl.ds(16 * j, 16)] = acc
    pltpu.sync_copy(hist_pub, xch_hbm.at[cid * 16 + sid])
    plsc.subcore_barrier()
    pltpu.sync_copy(xch_hbm.at[pl.ds(cid * 16, 16)], histv)

    for j in range(P // 16):
        tot = jnp.zeros((16,), jnp.int32)
        pre = jnp.zeros((16,), jnp.int32)
        for w2 in range(16):
            v = histv[w2, pl.ds(16 * j, 16)]
            tot = tot + v
            pre = pre + v * jnp.where(w2 < sid, 1, 0)
        for l in range(16):
            tots[16 * j + l] = tot[l]
            pres[16 * j + l] = pre[l]

    def obody(p, carry):
        t = tots[p]
        nb = lax.shift_right_logical(t + (BLK - 1), 7)
        offs[p] = carry * BLK + pres[p]
        blk0[p] = carry
        nbv[p] = nb
        return carry + nb

    totblk = lax.fori_loop(0, P, obody, 0)

    for j in range(NBLOCKS // 16):
        bpv[pl.ds(16 * j, 16)] = jnp.zeros((16,), jnp.int32)

    def bpbody(p, _):
        e = blk0[p] + nbv[p]
        for j in range(NBLOCKS // 16):
            bvec = lane + 16 * j
            sl = pl.ds(16 * j, 16)
            bpv[sl] = bpv[sl] + jnp.where(bvec >= e, 1, 0)
        return _

    lax.fori_loop(0, P, bpbody, 0)
    for j in range(NBLOCKS // 16):
        bvec = lane + 16 * j
        sl = pl.ds(16 * j, 16)
        bpv[sl] = jnp.where(bvec < totblk, bpv[sl], -1)

    @pl.when((sid == 0) & (cid == 0))
    def _bp():
        pltpu.sync_copy(bpv, bp_hbm)

    def pass2(cc, _):
        d0 = jnp.zeros((16,), jnp.int32)
        d1 = jnp.zeros((16,), jnp.int32)
        for j in range(16):
            i = cc * 16 + j
            a0 = la[2 * i]
            p0 = lax.shift_right_logical(a0, 12)
            r0 = jnp.bitwise_and(a0, 4095)
            d0 = jnp.where(lane == j, offs[p0] + r0, d0)
            a1 = la[2 * i + 1]
            p1 = lax.shift_right_logical(a1, 12)
            r1 = jnp.bitwise_and(a1, 4095)
            d1 = jnp.where(lane == j, offs[p1] + r1, d1)
        base = cc * 16
        inv0v[pl.ds(base, 16)] = d0
        inv1v[pl.ds(base, 16)] = d1
        h = lax.shift_right_logical(cc, 3)
        chunk = jnp.bitwise_and(lax.shift_right_logical(cc, 1), 3)
        colb = jnp.bitwise_and(cc, 1) * 16
        fb = h * 256 + chunk * 32 + colb
        idxvf[pl.ds(fb, 16)] = d0
        idxvf[pl.ds(fb + 128, 16)] = d1
        return _

    lax.fori_loop(0, 16, pass2, 0)

    @pl.when(cid == 0)
    def _wr():
        pltpu.sync_copy(inv0v, inv0_hbm.at[pl.ds(sid * tpt, tpt)])
        pltpu.sync_copy(inv1v, inv1_hbm.at[pl.ds(sid * tpt, tpt)])
        pltpu.sync_copy(idxvf, idx_hbm.at[pl.ds(sid * 512, 512)])


def _k2a(pp):
    mesh = plsc.VectorSubcoreMesh(core_axis_name="c", subcore_axis_name="s")
    tpt = T // 16
    f = functools.partial(
        pl.kernel,
        out_type=[
            jax.ShapeDtypeStruct((T,), jnp.int32),
            jax.ShapeDtypeStruct((T,), jnp.int32),
            jax.ShapeDtypeStruct((32 * 256,), jnp.int32),
            jax.ShapeDtypeStruct((NBLOCKS,), jnp.int32),
            jax.ShapeDtypeStruct((32, P), jnp.int32),
        ],
        mesh=mesh,
        scratch_types=[
            pltpu.VMEM((tpt, 128), jnp.int32),
            pltpu.VMEM((P,), jnp.int32),
            pltpu.VMEM((16, P), jnp.int32),
            pltpu.VMEM((tpt,), jnp.int32),
            pltpu.VMEM((tpt,), jnp.int32),
            pltpu.VMEM((512,), jnp.int32),
            pltpu.VMEM((NBLOCKS,), jnp.int32),
            pltpu.SMEM((P,), jnp.int32),
            pltpu.SMEM((2 * tpt,), jnp.int32),
            pltpu.SMEM((P,), jnp.int32),
            pltpu.SMEM((P,), jnp.int32),
            pltpu.SMEM((P,), jnp.int32),
            pltpu.SMEM((P,), jnp.int32),
            pltpu.SMEM((P,), jnp.int32),
        ],
    )(_k2a_body)
    return f(pp)


def _k2b_body(x_hbm, idx_hbm, xs_hbm, idxflat, idxv2, xrow, sem):
    v = lax.axis_index("s") * 2 + lax.axis_index("c")
    pltpu.sync_copy(idx_hbm.at[pl.ds(v * 256, 256)], idxflat)
    for r in range(8):
        for h in range(2):
            idxv2[r, pl.ds(h * 16, 16)] = idxflat[pl.ds(r * 32 + h * 16, 16)]
    for c in range(4):
        pltpu.sync_copy(x_hbm.at[pl.ds(v * 128 + c * 32, 32)], xrow)
        pltpu.async_copy(xrow, xs_hbm.at[idxv2.at[c]], sem).wait()
        pltpu.async_copy(xrow, xs_hbm.at[idxv2.at[4 + c]], sem).wait()


def _k2b(x, idx):
    mesh = plsc.VectorSubcoreMesh(core_axis_name="c", subcore_axis_name="s")
    f = functools.partial(
        pl.kernel,
        out_type=[jax.ShapeDtypeStruct((NROWS, D), _F32)],
        mesh=mesh,
        scratch_types=[
            pltpu.VMEM((256,), jnp.int32),
            pltpu.VMEM((8, 32), jnp.int32),
            pltpu.VMEM((32, D), _F32),
            pltpu.SemaphoreType.DMA,
        ],
    )(_k2b_body)
    return f(x, idx)[0]


def _k3_body(bp_ref, xs_ref, down_ref, ce_ref, ceb_ref, cd_ref, up_ref,
             out_ref):
    b = pl.program_id(0)

    @pl.when(bp_ref[b] >= 0)
    def _():
        iota_c = jax.lax.broadcasted_iota(jnp.int32, (BLK, C), 1).astype(_F32)
        xb = xs_ref[:]
        xsub = jnp.dot(xb, down_ref[0], preferred_element_type=_F32)
        cl = jnp.dot(xsub, ce_ref[0],
                     preferred_element_type=_F32) + ceb_ref[0]
        mask, _, _ = _top2(cl, iota_c)
        codes = jnp.where(mask, 1.0, 0.0)
        csub = jnp.dot(codes, cd_ref[0], preferred_element_type=_F32)
        out_ref[:] = jnp.dot(csub, up_ref[0], preferred_element_type=_F32)


def _k3(bp, xs, down_t, ce_t, ce_b3, cd_t, up_t):
    grid_spec = pltpu.PrefetchScalarGridSpec(
        num_scalar_prefetch=1,
        grid=(NBLOCKS,),
        in_specs=[
            pl.BlockSpec((BLK, D), lambda b, bp: (b, 0)),
            pl.BlockSpec((1, D, S), lambda b, bp: (jnp.maximum(bp[b], 0), 0, 0)),
            pl.BlockSpec((1, S, C), lambda b, bp: (jnp.maximum(bp[b], 0), 0, 0)),
            pl.BlockSpec((1, 1, C), lambda b, bp: (jnp.maximum(bp[b], 0), 0, 0)),
            pl.BlockSpec((1, C, S), lambda b, bp: (jnp.maximum(bp[b], 0), 0, 0)),
            pl.BlockSpec((1, S, D), lambda b, bp: (jnp.maximum(bp[b], 0), 0, 0)),
        ],
        out_specs=pl.BlockSpec((BLK, D), lambda b, bp: (b, 0)),
    )
    return pl.pallas_call(
        _k3_body,
        grid_spec=grid_spec,
        out_shape=jax.ShapeDtypeStruct((NROWS, D), _F32),
    )(bp, xs, down_t, ce_t, ce_b3, cd_t, up_t)


def _k4_body(rows_hbm, base_hbm, inv0_hbm, inv1_hbm, out_hbm,
             inv0v, inv1v, buf0, buf1, bb, sem):
    v = lax.axis_index("s") * 2 + lax.axis_index("c")
    t0 = v * 128
    pltpu.sync_copy(inv0_hbm.at[pl.ds(t0, 128)], inv0v)
    pltpu.sync_copy(inv1_hbm.at[pl.ds(t0, 128)], inv1v)
    for c in range(4):
        pltpu.async_copy(rows_hbm.at[inv0v.at[pl.ds(c * 32, 32)]],
                         buf0, sem).wait()
        pltpu.async_copy(rows_hbm.at[inv1v.at[pl.ds(c * 32, 32)]],
                         buf1, sem).wait()
        pltpu.sync_copy(base_hbm.at[pl.ds(t0 + c * 32, 32)], bb)

        def radd(r, _):
            for h in range(D // 16):
                sl = pl.ds(h * 16, 16)
                bb[r, sl] = bb[r, sl] + buf0[r, sl] + buf1[r, sl]
            return _

        lax.fori_loop(0, 32, radd, 0)
        pltpu.sync_copy(bb, out_hbm.at[pl.ds(t0 + c * 32, 32)])


def _k4(rows, base, inv0, inv1):
    mesh = plsc.VectorSubcoreMesh(core_axis_name="c", subcore_axis_name="s")
    f = functools.partial(
        pl.kernel,
        out_type=[jax.ShapeDtypeStruct((T, D), _F32)],
        mesh=mesh,
        scratch_types=[
            pltpu.VMEM((128,), jnp.int32),
            pltpu.VMEM((128,), jnp.int32),
            pltpu.VMEM((32, D), _F32),
            pltpu.VMEM((32, D), _F32),
            pltpu.VMEM((32, D), _F32),
            pltpu.SemaphoreType.DMA,
        ],
    )(_k4_body)
    return f(rows, base, inv0, inv1)[0]


def kernel(x, pe_w, pe_b, pd_w, down_w, up_w, ce_w, ce_b, cd_w, dec_bias):
    down_t = jnp.transpose(down_w, (0, 2, 1))
    ce_t = jnp.transpose(ce_w, (0, 2, 1))
    ce_b3 = ce_b.reshape(P, 1, C)
    cd_t = jnp.transpose(cd_w, (0, 2, 1))
    up_t = jnp.transpose(up_w, (0, 2, 1))

    base, pp = _k1(x, pe_w, pe_b, pd_w, dec_bias)
    inv0, inv1, idx, bp, _ = _k2a(pp)
    xs = _k2b(x, idx)
    rows = _k3(bp, xs, down_t, ce_t, ce_b3, cd_t, up_t)
    return _k4(rows, base, inv0, inv1)


# speedup vs baseline: 23.6202x; 1.0706x over previous
import functools

import jax
import jax.numpy as jnp
from jax import lax
from jax.experimental import pallas as pl
from jax.experimental.pallas import tpu as pltpu
from jax.experimental.pallas import tpu_sc as plsc

D = 768
P = 64
S = 32
C = 128
T = 4096
BLK = 128
NBLOCKS = 128
NROWS = NBLOCKS * BLK
BT1 = 512

_NEG = -3.0e38
_F32 = jnp.float32


def _top2(logits, iota_f):
    big = 1.0e9
    m1 = jnp.max(logits, axis=1, keepdims=True)
    i1 = jnp.min(jnp.where(logits == m1, iota_f, big), axis=1, keepdims=True)
    sel1 = iota_f == i1
    rest = jnp.where(sel1, _NEG, logits)
    m2 = jnp.max(rest, axis=1, keepdims=True)
    i2 = jnp.min(jnp.where(rest == m2, iota_f, big), axis=1, keepdims=True)
    return jnp.logical_or(sel1, iota_f == i2), i1, i2


def _k1_body(x_ref, pewt_ref, peb_ref, pdw_ref, bias_ref, base_ref, pp_ref):
    xb = x_ref[:]
    iota_p = jax.lax.broadcasted_iota(jnp.int32, (BT1, P), 1).astype(_F32)
    logits = jnp.dot(xb, pewt_ref[:], preferred_element_type=_F32) + peb_ref[:]
    mask, i1, i2 = _top2(logits, iota_p)
    pcodes = jnp.where(mask, 1.0, 0.0)
    base_ref[:] = jnp.dot(pcodes, pdw_ref[:],
                          preferred_element_type=_F32) + bias_ref[:]
    packed = i1 * 64.0 + i2
    pp_ref[:] = jnp.broadcast_to(packed, (BT1, 128)).astype(jnp.int32)


def _k1(x, pe_w, pe_b, pd_w, dec_bias):
    const = lambda *dims: pl.BlockSpec(dims, lambda i: (0,) * len(dims))
    return pl.pallas_call(
        _k1_body,
        grid=(T // BT1,),
        in_specs=[
            pl.BlockSpec((BT1, D), lambda i: (i, 0)),
            const(D, P),
            const(1, P),
            const(P, D),
            const(1, D),
        ],
        out_specs=[
            pl.BlockSpec((BT1, D), lambda i: (i, 0)),
            pl.BlockSpec((BT1, 128), lambda i: (i, 0)),
        ],
        out_shape=[
            jax.ShapeDtypeStruct((T, D), _F32),
            jax.ShapeDtypeStruct((T, 128), jnp.int32),
        ],
    )(x, pe_w.T, pe_b.reshape(1, P), pd_w, dec_bias.reshape(1, D))


def _k2a_body(pp_hbm, inv0_hbm, inv1_hbm, idx_hbm, bp_hbm, xch_hbm,
              ppv, hist_pub, histv, inv0v, inv1v, idxvf, bpv,
              runcnt, la, tots, pres, offs, blk0, nbv):
    sid = lax.axis_index("s")
    cid = lax.axis_index("c")
    tpt = T // 16
    lane = lax.iota(jnp.int32, 16)
    pltpu.sync_copy(pp_hbm.at[pl.ds(sid * tpt, tpt)], ppv)

    def zbody(p, _):
        runcnt[p] = 0
        return _

    lax.fori_loop(0, P, zbody, 0)

    def pass1(cc, _):
        for j in range(16):
            i = cc * 16 + j
            pk = ppv[i, pl.ds(0, 16)][0]
            p1 = lax.shift_right_logical(pk, 6)
            p2 = jnp.bitwise_and(pk, 63)
            r1 = runcnt[p1]
            la[2 * i] = p1 * 4096 + r1
            runcnt[p1] = r1 + 1
            r2 = runcnt[p2]
            la[2 * i + 1] = p2 * 4096 + r2
            runcnt[p2] = r2 + 1
        return _

    lax.fori_loop(0, 16, pass1, 0)

    for j in range(P // 16):
        acc = jnp.zeros((16,), jnp.int32)
        for l in range(16):
            acc = jnp.where(lane == l, runcnt[16 * j + l], acc)
        hist_pub[pl.ds(16 * j, 16)] = acc
    pltpu.sync_copy(hist_pub, xch_hbm.at[cid * 16 + sid])
    plsc.subcore_barrier()
    pltpu.sync_copy(xch_hbm.at[pl.ds(cid * 16, 16)], histv)

    for j in range(P // 16):
        tot = jnp.zeros((16,), jnp.int32)
        pre = jnp.zeros((16,), jnp.int32)
        for w2 in range(16):
            v = histv[w2, pl.ds(16 * j, 16)]
            tot = tot + v
            pre = pre + v * jnp.where(w2 < sid, 1, 0)
        for l in range(16):
            tots[16 * j + l] = tot[l]
            pres[16 * j + l] = pre[l]

    def obody(p, carry):
        t = tots[p]
        nb = lax.shift_right_logical(t + (BLK - 1), 7)
        offs[p] = carry * BLK + pres[p]
        blk0[p] = carry
        nbv[p] = nb
        return carry + nb

    totblk = lax.fori_loop(0, P, obody, 0)

    for j in range(NBLOCKS // 16):
        bpv[pl.ds(16 * j, 16)] = jnp.zeros((16,), jnp.int32)

    def bpbody(p, _):
        e = blk0[p] + nbv[p]
        for j in range(NBLOCKS // 16):
            bvec = lane + 16 * j
            sl = pl.ds(16 * j, 16)
            bpv[sl] = bpv[sl] + jnp.where(bvec >= e, 1, 0)
        return _

    lax.fori_loop(0, P, bpbody, 0)
    for j in range(NBLOCKS // 16):
        bvec = lane + 16 * j
        sl = pl.ds(16 * j, 16)
        bpv[sl] = jnp.where(bvec < totblk, bpv[sl], -1)

    @pl.when((sid == 0) & (cid == 0))
    def _bp():
        pltpu.sync_copy(bpv, bp_hbm)

    def pass2(cc, _):
        d0 = jnp.zeros((16,), jnp.int32)
        d1 = jnp.zeros((16,), jnp.int32)
        for j in range(16):
            i = cc * 16 + j
            a0 = la[2 * i]
            p0 = lax.shift_right_logical(a0, 12)
            r0 = jnp.bitwise_and(a0, 4095)
            d0 = jnp.where(lane == j, offs[p0] + r0, d0)
            a1 = la[2 * i + 1]
            p1 = lax.shift_right_logical(a1, 12)
            r1 = jnp.bitwise_and(a1, 4095)
            d1 = jnp.where(lane == j, offs[p1] + r1, d1)
        base = cc * 16
        inv0v[pl.ds(base, 16)] = d0
        inv1v[pl.ds(base, 16)] = d1
        h = lax.shift_right_logical(cc, 3)
        chunk = jnp.bitwise_and(lax.shift_right_logical(cc, 1), 3)
        colb = jnp.bitwise_and(cc, 1) * 16
        fb = h * 256 + chunk * 32 + colb
        idxvf[pl.ds(fb, 16)] = d0
        idxvf[pl.ds(fb + 128, 16)] = d1
        return _

    lax.fori_loop(0, 16, pass2, 0)

    @pl.when(cid == 0)
    def _wr():
        pltpu.sync_copy(inv0v, inv0_hbm.at[pl.ds(sid * tpt, tpt)])
        pltpu.sync_copy(inv1v, inv1_hbm.at[pl.ds(sid * tpt, tpt)])
        pltpu.sync_copy(idxvf, idx_hbm.at[pl.ds(sid * 512, 512)])


def _k2a(pp):
    mesh = plsc.VectorSubcoreMesh(core_axis_name="c", subcore_axis_name="s")
    tpt = T // 16
    f = functools.partial(
        pl.kernel,
        out_type=[
            jax.ShapeDtypeStruct((T,), jnp.int32),
            jax.ShapeDtypeStruct((T,), jnp.int32),
            jax.ShapeDtypeStruct((32 * 256,), jnp.int32),
            jax.ShapeDtypeStruct((NBLOCKS,), jnp.int32),
            jax.ShapeDtypeStruct((32, P), jnp.int32),
        ],
        mesh=mesh,
        scratch_types=[
            pltpu.VMEM((tpt, 128), jnp.int32),
            pltpu.VMEM((P,), jnp.int32),
            pltpu.VMEM((16, P), jnp.int32),
            pltpu.VMEM((tpt,), jnp.int32),
            pltpu.VMEM((tpt,), jnp.int32),
            pltpu.VMEM((512,), jnp.int32),
            pltpu.VMEM((NBLOCKS,), jnp.int32),
            pltpu.SMEM((P,), jnp.int32),
            pltpu.SMEM((2 * tpt,), jnp.int32),
            pltpu.SMEM((P,), jnp.int32),
            pltpu.SMEM((P,), jnp.int32),
            pltpu.SMEM((P,), jnp.int32),
            pltpu.SMEM((P,), jnp.int32),
            pltpu.SMEM((P,), jnp.int32),
        ],
    )(_k2a_body)
    return f(pp)


def _k2b_body(x_hbm, idx_hbm, xs_hbm, idxflat, idxv2, xrow, sem):
    v = lax.axis_index("s") * 2 + lax.axis_index("c")
    pltpu.sync_copy(idx_hbm.at[pl.ds(v * 256, 256)], idxflat)
    for r in range(8):
        for h in range(2):
            idxv2[r, pl.ds(h * 16, 16)] = idxflat[pl.ds(r * 32 + h * 16, 16)]
    for c in range(4):
        pltpu.sync_copy(x_hbm.at[pl.ds(v * 128 + c * 32, 32)], xrow)
        pltpu.async_copy(xrow, xs_hbm.at[idxv2.at[c]], sem).wait()
        pltpu.async_copy(xrow, xs_hbm.at[idxv2.at[4 + c]], sem).wait()


def _k2b(x, idx):
    mesh = plsc.VectorSubcoreMesh(core_axis_name="c", subcore_axis_name="s")
    f = functools.partial(
        pl.kernel,
        out_type=[jax.ShapeDtypeStruct((NROWS, D), _F32)],
        mesh=mesh,
        scratch_types=[
            pltpu.VMEM((256,), jnp.int32),
            pltpu.VMEM((8, 32), jnp.int32),
            pltpu.VMEM((32, D), _F32),
            pltpu.SemaphoreType.DMA,
        ],
    )(_k2b_body)
    return f(x, idx)[0]


def _k3_body(bp_ref, xs_ref, down_ref, ce_ref, ceb_ref, cd_ref, up_ref,
             out_ref):
    b = pl.program_id(0)

    @pl.when(bp_ref[b] >= 0)
    def _():
        iota_c = jax.lax.broadcasted_iota(jnp.int32, (BLK, C), 1).astype(_F32)
        xb = xs_ref[:]
        xsub = jnp.dot(xb, down_ref[0], preferred_element_type=_F32)
        cl = jnp.dot(xsub, ce_ref[0],
                     preferred_element_type=_F32) + ceb_ref[0]
        mask, _, _ = _top2(cl, iota_c)
        codes = jnp.where(mask, 1.0, 0.0)
        csub = jnp.dot(codes, cd_ref[0], preferred_element_type=_F32)
        out_ref[:] = jnp.dot(csub, up_ref[0], preferred_element_type=_F32)


def _k3(bp, xs, down_t, ce_t, ce_b3, cd_t, up_t):
    wmap = lambda b, bp: (jnp.maximum(bp[b], 0), 0, 0)
    grid_spec = pltpu.PrefetchScalarGridSpec(
        num_scalar_prefetch=1,
        grid=(NBLOCKS,),
        in_specs=[
            pl.BlockSpec((BLK, D),
                         lambda b, bp: (jnp.where(bp[b] >= 0, b, 0), 0)),
            pl.BlockSpec((1, D, S), wmap),
            pl.BlockSpec((1, S, C), wmap),
            pl.BlockSpec((1, 1, C), wmap),
            pl.BlockSpec((1, C, S), wmap),
            pl.BlockSpec((1, S, D), wmap),
        ],
        out_specs=pl.BlockSpec(
            (BLK, D), lambda b, bp: (jnp.where(bp[b] >= 0, b, NBLOCKS), 0)),
    )
    return pl.pallas_call(
        _k3_body,
        grid_spec=grid_spec,
        out_shape=jax.ShapeDtypeStruct((NROWS + BLK, D), _F32),
    )(bp, xs, down_t, ce_t, ce_b3, cd_t, up_t)


def _k4_body(rows_hbm, base_hbm, inv0_hbm, inv1_hbm, out_hbm,
             inv0v, inv1v, buf0, buf1, bb, sem):
    v = lax.axis_index("s") * 2 + lax.axis_index("c")
    t0 = v * 128
    pltpu.sync_copy(inv0_hbm.at[pl.ds(t0, 128)], inv0v)
    pltpu.sync_copy(inv1_hbm.at[pl.ds(t0, 128)], inv1v)
    for c in range(4):
        pltpu.async_copy(rows_hbm.at[inv0v.at[pl.ds(c * 32, 32)]],
                         buf0, sem).wait()
        pltpu.async_copy(rows_hbm.at[inv1v.at[pl.ds(c * 32, 32)]],
                         buf1, sem).wait()
        pltpu.sync_copy(base_hbm.at[pl.ds(t0 + c * 32, 32)], bb)

        def radd(r, _):
            for h in range(D // 16):
                sl = pl.ds(h * 16, 16)
                bb[r, sl] = bb[r, sl] + buf0[r, sl] + buf1[r, sl]
            return _

        lax.fori_loop(0, 32, radd, 0)
        pltpu.sync_copy(bb, out_hbm.at[pl.ds(t0 + c * 32, 32)])


def _k4(rows, base, inv0, inv1):
    mesh = plsc.VectorSubcoreMesh(core_axis_name="c", subcore_axis_name="s")
    f = functools.partial(
        pl.kernel,
        out_type=[jax.ShapeDtypeStruct((T, D), _F32)],
        mesh=mesh,
        scratch_types=[
            pltpu.VMEM((128,), jnp.int32),
            pltpu.VMEM((128,), jnp.int32),
            pltpu.VMEM((32, D), _F32),
            pltpu.VMEM((32, D), _F32),
            pltpu.VMEM((32, D), _F32),
            pltpu.SemaphoreType.DMA,
        ],
    )(_k4_body)
    return f(rows, base, inv0, inv1)[0]


def kernel(x, pe_w, pe_b, pd_w, down_w, up_w, ce_w, ce_b, cd_w, dec_bias):
    down_t = jnp.transpose(down_w, (0, 2, 1))
    ce_t = jnp.transpose(ce_w, (0, 2, 1))
    ce_b3 = ce_b.reshape(P, 1, C)
    cd_t = jnp.transpose(cd_w, (0, 2, 1))
    up_t = jnp.transpose(up_w, (0, 2, 1))

    base, pp = _k1(x, pe_w, pe_b, pd_w, dec_bias)
    inv0, inv1, idx, bp, _ = _k2a(pp)
    xs = _k2b(x, idx)
    rows = _k3(bp, xs, down_t, ce_t, ce_b3, cd_t, up_t)
    return _k4(rows, base, inv0, inv1)


# speedup vs baseline: 25.0743x; 1.0616x over previous
import functools

import jax
import jax.numpy as jnp
from jax import lax
from jax.experimental import pallas as pl
from jax.experimental.pallas import tpu as pltpu
from jax.experimental.pallas import tpu_sc as plsc

D = 768
P = 64
S = 32
C = 128
T = 4096
BLK = 128
NBLOCKS = 128
NROWS = NBLOCKS * BLK
BT1 = 512

_NEG = -3.0e38
_F32 = jnp.float32


def _top2(logits, iota_f):
    big = 1.0e9
    m1 = jnp.max(logits, axis=1, keepdims=True)
    i1 = jnp.min(jnp.where(logits == m1, iota_f, big), axis=1, keepdims=True)
    sel1 = iota_f == i1
    rest = jnp.where(sel1, _NEG, logits)
    m2 = jnp.max(rest, axis=1, keepdims=True)
    i2 = jnp.min(jnp.where(rest == m2, iota_f, big), axis=1, keepdims=True)
    return jnp.logical_or(sel1, iota_f == i2), i1, i2


def _k1_body(x_ref, pewt_ref, peb_ref, pdw_ref, bias_ref, base_ref, pp_ref):
    xb = x_ref[:]
    iota_p = jax.lax.broadcasted_iota(jnp.int32, (BT1, P), 1).astype(_F32)
    logits = jnp.dot(xb, pewt_ref[:], preferred_element_type=_F32) + peb_ref[:]
    mask, i1, i2 = _top2(logits, iota_p)
    pcodes = jnp.where(mask, 1.0, 0.0)
    base_ref[:] = jnp.dot(pcodes, pdw_ref[:],
                          preferred_element_type=_F32) + bias_ref[:]
    packed = i1 * 64.0 + i2
    pp_ref[:] = jnp.broadcast_to(packed, (BT1, 128)).astype(jnp.int32)


def _k1(x, pe_w, pe_b, pd_w, dec_bias):
    const = lambda *dims: pl.BlockSpec(dims, lambda i: (0,) * len(dims))
    return pl.pallas_call(
        _k1_body,
        grid=(T // BT1,),
        in_specs=[
            pl.BlockSpec((BT1, D), lambda i: (i, 0)),
            const(D, P),
            const(1, P),
            const(P, D),
            const(1, D),
        ],
        out_specs=[
            pl.BlockSpec((BT1, D), lambda i: (i, 0)),
            pl.BlockSpec((BT1, 128), lambda i: (i, 0)),
        ],
        out_shape=[
            jax.ShapeDtypeStruct((T, D), _F32),
            jax.ShapeDtypeStruct((T, 128), jnp.int32),
        ],
    )(x, pe_w.T, pe_b.reshape(1, P), pd_w, dec_bias.reshape(1, D))


def _k2a_body(pp_hbm, inv0_hbm, inv1_hbm, idx_hbm, bp_hbm, xch_hbm,
              ppv, hist_pub, histv, inv0v, inv1v, idxvf, bpv,
              runcnt, la, tots, pres, offs, blk0, nbv):
    sid = lax.axis_index("s")
    cid = lax.axis_index("c")
    tpt = T // 16
    lane = lax.iota(jnp.int32, 16)
    pltpu.sync_copy(pp_hbm.at[pl.ds(sid * tpt, tpt)], ppv)

    def zbody(p, _):
        runcnt[p] = 0
        return _

    lax.fori_loop(0, P, zbody, 0)

    def pass1(cc, _):
        for j in range(16):
            i = cc * 16 + j
            pk = ppv[i, pl.ds(0, 16)][0]
            p1 = lax.shift_right_logical(pk, 6)
            p2 = jnp.bitwise_and(pk, 63)
            r1 = runcnt[p1]
            la[2 * i] = p1 * 4096 + r1
            runcnt[p1] = r1 + 1
            r2 = runcnt[p2]
            la[2 * i + 1] = p2 * 4096 + r2
            runcnt[p2] = r2 + 1
        return _

    lax.fori_loop(0, 16, pass1, 0)

    for j in range(P // 16):
        acc = jnp.zeros((16,), jnp.int32)
        for l in range(16):
            acc = jnp.where(lane == l, runcnt[16 * j + l], acc)
        hist_pub[pl.ds(16 * j, 16)] = acc
    pltpu.sync_copy(hist_pub, xch_hbm.at[cid * 16 + sid])
    plsc.subcore_barrier()
    pltpu.sync_copy(xch_hbm.at[pl.ds(cid * 16, 16)], histv)

    for j in range(P // 16):
        tot = jnp.zeros((16,), jnp.int32)
        pre = jnp.zeros((16,), jnp.int32)
        for w2 in range(16):
            v = histv[w2, pl.ds(16 * j, 16)]
            tot = tot + v
            pre = pre + v * jnp.where(w2 < sid, 1, 0)
        for l in range(16):
            tots[16 * j + l] = tot[l]
            pres[16 * j + l] = pre[l]

    def obody(p, carry):
        t = tots[p]
        nb = lax.shift_right_logical(t + (BLK - 1), 7)
        offs[p] = carry * BLK + pres[p]
        blk0[p] = carry
        nbv[p] = nb
        return carry + nb

    totblk = lax.fori_loop(0, P, obody, 0)

    for j in range(NBLOCKS // 16):
        bpv[pl.ds(16 * j, 16)] = jnp.zeros((16,), jnp.int32)

    def bpbody(p, _):
        e = blk0[p] + nbv[p]
        for j in range(NBLOCKS // 16):
            bvec = lane + 16 * j
            sl = pl.ds(16 * j, 16)
            bpv[sl] = bpv[sl] + jnp.where(bvec >= e, 1, 0)
        return _

    lax.fori_loop(0, P, bpbody, 0)
    for j in range(NBLOCKS // 16):
        bvec = lane + 16 * j
        sl = pl.ds(16 * j, 16)
        bpv[sl] = jnp.where(bvec < totblk, bpv[sl], -1)

    @pl.when((sid == 0) & (cid == 0))
    def _bp():
        pltpu.sync_copy(bpv, bp_hbm)

    def pass2(cc, _):
        d0 = jnp.zeros((16,), jnp.int32)
        d1 = jnp.zeros((16,), jnp.int32)
        for j in range(16):
            i = cc * 16 + j
            a0 = la[2 * i]
            p0 = lax.shift_right_logical(a0, 12)
            r0 = jnp.bitwise_and(a0, 4095)
            d0 = jnp.where(lane == j, offs[p0] + r0, d0)
            a1 = la[2 * i + 1]
            p1 = lax.shift_right_logical(a1, 12)
            r1 = jnp.bitwise_and(a1, 4095)
            d1 = jnp.where(lane == j, offs[p1] + r1, d1)
        base = cc * 16
        inv0v[pl.ds(base, 16)] = d0
        inv1v[pl.ds(base, 16)] = d1
        h = lax.shift_right_logical(cc, 3)
        chunk = jnp.bitwise_and(lax.shift_right_logical(cc, 1), 3)
        colb = jnp.bitwise_and(cc, 1) * 16
        fb = h * 256 + chunk * 32 + colb
        idxvf[pl.ds(fb, 16)] = d0
        idxvf[pl.ds(fb + 128, 16)] = d1
        return _

    lax.fori_loop(0, 16, pass2, 0)

    @pl.when(cid == 0)
    def _wr():
        pltpu.sync_copy(inv0v, inv0_hbm.at[pl.ds(sid * tpt, tpt)])
        pltpu.sync_copy(inv1v, inv1_hbm.at[pl.ds(sid * tpt, tpt)])
        pltpu.sync_copy(idxvf, idx_hbm.at[pl.ds(sid * 512, 512)])


def _k2a(pp):
    mesh = plsc.VectorSubcoreMesh(core_axis_name="c", subcore_axis_name="s")
    tpt = T // 16
    f = functools.partial(
        pl.kernel,
        out_type=[
            jax.ShapeDtypeStruct((T,), jnp.int32),
            jax.ShapeDtypeStruct((T,), jnp.int32),
            jax.ShapeDtypeStruct((32 * 256,), jnp.int32),
            jax.ShapeDtypeStruct((NBLOCKS,), jnp.int32),
            jax.ShapeDtypeStruct((32, P), jnp.int32),
        ],
        mesh=mesh,
        scratch_types=[
            pltpu.VMEM((tpt, 128), jnp.int32),
            pltpu.VMEM((P,), jnp.int32),
            pltpu.VMEM((16, P), jnp.int32),
            pltpu.VMEM((tpt,), jnp.int32),
            pltpu.VMEM((tpt,), jnp.int32),
            pltpu.VMEM((512,), jnp.int32),
            pltpu.VMEM((NBLOCKS,), jnp.int32),
            pltpu.SMEM((P,), jnp.int32),
            pltpu.SMEM((2 * tpt,), jnp.int32),
            pltpu.SMEM((P,), jnp.int32),
            pltpu.SMEM((P,), jnp.int32),
            pltpu.SMEM((P,), jnp.int32),
            pltpu.SMEM((P,), jnp.int32),
            pltpu.SMEM((P,), jnp.int32),
        ],
    )(_k2a_body)
    return f(pp)


def _k2b_body(x_hbm, idx_hbm, xs_hbm, idxflat, idxv2, xrow, sem):
    v = lax.axis_index("s") * 2 + lax.axis_index("c")
    pltpu.sync_copy(idx_hbm.at[pl.ds(v * 256, 256)], idxflat)
    for r in range(8):
        for h in range(2):
            idxv2[r, pl.ds(h * 16, 16)] = idxflat[pl.ds(r * 32 + h * 16, 16)]
    for c in range(4):
        pltpu.sync_copy(x_hbm.at[pl.ds(v * 128 + c * 32, 32)], xrow)
        pltpu.async_copy(xrow, xs_hbm.at[idxv2.at[c]], sem).wait()
        pltpu.async_copy(xrow, xs_hbm.at[idxv2.at[4 + c]], sem).wait()


def _k2b(x, idx):
    mesh = plsc.VectorSubcoreMesh(core_axis_name="c", subcore_axis_name="s")
    f = functools.partial(
        pl.kernel,
        out_type=[jax.ShapeDtypeStruct((NROWS, D), _F32)],
        mesh=mesh,
        scratch_types=[
            pltpu.VMEM((256,), jnp.int32),
            pltpu.VMEM((8, 32), jnp.int32),
            pltpu.VMEM((32, D), _F32),
            pltpu.SemaphoreType.DMA,
        ],
    )(_k2b_body)
    return f(x, idx)[0]


def _k3_body(bp_ref, xs_ref, down_ref, ce_ref, ceb_ref, cd_ref, up_ref,
             out_ref):
    b = pl.program_id(0)

    @pl.when(bp_ref[b] >= 0)
    def _():
        iota_c = jax.lax.broadcasted_iota(jnp.int32, (BLK, C), 1).astype(_F32)
        xb = xs_ref[:]
        xsub = jnp.dot(xb, down_ref[0], preferred_element_type=_F32)
        cl = jnp.dot(xsub, ce_ref[0],
                     preferred_element_type=_F32) + ceb_ref[0]
        mask, _, _ = _top2(cl, iota_c)
        codes = jnp.where(mask, 1.0, 0.0)
        csub = jnp.dot(codes, cd_ref[0], preferred_element_type=_F32)
        out_ref[:] = jnp.dot(csub, up_ref[0], preferred_element_type=_F32)


def _k3(bp, xs, down_t, ce_t, ce_b3, cd_t, up_t):
    wmap = lambda b, bp: (jnp.maximum(bp[b], 0), 0, 0)
    grid_spec = pltpu.PrefetchScalarGridSpec(
        num_scalar_prefetch=1,
        grid=(NBLOCKS,),
        in_specs=[
            pl.BlockSpec((BLK, D),
                         lambda b, bp: (jnp.where(bp[b] >= 0, b, 0), 0)),
            pl.BlockSpec((1, D, S), wmap),
            pl.BlockSpec((1, S, C), wmap),
            pl.BlockSpec((1, 1, C), wmap),
            pl.BlockSpec((1, C, S), wmap),
            pl.BlockSpec((1, S, D), wmap),
        ],
        out_specs=pl.BlockSpec(
            (BLK, D), lambda b, bp: (jnp.where(bp[b] >= 0, b, NBLOCKS), 0)),
    )
    return pl.pallas_call(
        _k3_body,
        grid_spec=grid_spec,
        out_shape=jax.ShapeDtypeStruct((NROWS + BLK, D), _F32),
    )(bp, xs, down_t, ce_t, ce_b3, cd_t, up_t)


def _k4_body(rows_hbm, base_hbm, inv0_hbm, inv1_hbm, out_hbm,
             inv0v, inv1v, b0a, b0b, b1a, b1b, bba, bbb, s0, s1):
    v = lax.axis_index("s") * 2 + lax.axis_index("c")
    t0 = v * 128
    bufs0, bufs1, bbs, sems = [b0a, b0b], [b1a, b1b], [bba, bbb], [s0, s1]
    pltpu.sync_copy(inv0_hbm.at[pl.ds(t0, 128)], inv0v)
    pltpu.sync_copy(inv1_hbm.at[pl.ds(t0, 128)], inv1v)

    def start(c, slot):
        sl = pl.ds(c * 16, 16)
        return (
            pltpu.async_copy(rows_hbm.at[inv0v.at[sl]], bufs0[slot],
                             sems[slot]),
            pltpu.async_copy(rows_hbm.at[inv1v.at[sl]], bufs1[slot],
                             sems[slot]),
            pltpu.async_copy(base_hbm.at[pl.ds(t0 + c * 16, 16)], bbs[slot],
                             sems[slot]),
        )

    pend = start(0, 0)
    for c in range(8):
        slot = c & 1
        if c < 7:
            nxt = start(c + 1, 1 - slot)
        for dsc in pend:
            dsc.wait()
        bb, buf0, buf1 = bbs[slot], bufs0[slot], bufs1[slot]

        def radd(r, _):
            for h in range(D // 16):
                sl = pl.ds(h * 16, 16)
                bb[r, sl] = bb[r, sl] + buf0[r, sl] + buf1[r, sl]
            return _

        lax.fori_loop(0, 16, radd, 0)
        pltpu.sync_copy(bb, out_hbm.at[pl.ds(t0 + c * 16, 16)])
        if c < 7:
            pend = nxt


def _k4(rows, base, inv0, inv1):
    mesh = plsc.VectorSubcoreMesh(core_axis_name="c", subcore_axis_name="s")
    f = functools.partial(
        pl.kernel,
        out_type=[jax.ShapeDtypeStruct((T, D), _F32)],
        mesh=mesh,
        scratch_types=[
            pltpu.VMEM((128,), jnp.int32),
            pltpu.VMEM((128,), jnp.int32),
            pltpu.VMEM((16, D), _F32),
            pltpu.VMEM((16, D), _F32),
            pltpu.VMEM((16, D), _F32),
            pltpu.VMEM((16, D), _F32),
            pltpu.VMEM((16, D), _F32),
            pltpu.VMEM((16, D), _F32),
            pltpu.SemaphoreType.DMA,
            pltpu.SemaphoreType.DMA,
        ],
    )(_k4_body)
    return f(rows, base, inv0, inv1)[0]


def kernel(x, pe_w, pe_b, pd_w, down_w, up_w, ce_w, ce_b, cd_w, dec_bias):
    down_t = jnp.transpose(down_w, (0, 2, 1))
    ce_t = jnp.transpose(ce_w, (0, 2, 1))
    ce_b3 = ce_b.reshape(P, 1, C)
    cd_t = jnp.transpose(cd_w, (0, 2, 1))
    up_t = jnp.transpose(up_w, (0, 2, 1))

    base, pp = _k1(x, pe_w, pe_b, pd_w, dec_bias)
    inv0, inv1, idx, bp, _ = _k2a(pp)
    xs = _k2b(x, idx)
    rows = _k3(bp, xs, down_t, ce_t, ce_b3, cd_t, up_t)
    return _k4(rows, base, inv0, inv1)


# speedup vs baseline: 27.4434x; 1.0945x over previous
import functools

import jax
import jax.numpy as jnp
from jax import lax
from jax.experimental import pallas as pl
from jax.experimental.pallas import tpu as pltpu
from jax.experimental.pallas import tpu_sc as plsc

D = 768
P = 64
S = 32
C = 128
T = 4096
BLK = 128
NBLOCKS = 128
NROWS = NBLOCKS * BLK
BT1 = 512

_NEG = -3.0e38
_F32 = jnp.float32


def _top2(logits, iota_f):
    big = 1.0e9
    m1 = jnp.max(logits, axis=1, keepdims=True)
    i1 = jnp.min(jnp.where(logits == m1, iota_f, big), axis=1, keepdims=True)
    sel1 = iota_f == i1
    rest = jnp.where(sel1, _NEG, logits)
    m2 = jnp.max(rest, axis=1, keepdims=True)
    i2 = jnp.min(jnp.where(rest == m2, iota_f, big), axis=1, keepdims=True)
    return jnp.logical_or(sel1, iota_f == i2), i1, i2


def _k1_body(x_ref, pewt_ref, peb_ref, pdw_ref, bias_ref, base_ref, pp_ref):
    xb = x_ref[:]
    iota_p = jax.lax.broadcasted_iota(jnp.int32, (BT1, P), 1).astype(_F32)
    logits = jnp.dot(xb, pewt_ref[:], preferred_element_type=_F32) + peb_ref[:]
    mask, i1, i2 = _top2(logits, iota_p)
    pcodes = jnp.where(mask, 1.0, 0.0)
    base_ref[:] = jnp.dot(pcodes, pdw_ref[:],
                          preferred_element_type=_F32) + bias_ref[:]
    packed = i1 * 64.0 + i2
    pp_ref[:] = jnp.broadcast_to(packed, (BT1, 128)).astype(jnp.int32)


def _k1(x, pe_w, pe_b, pd_w, dec_bias):
    const = lambda *dims: pl.BlockSpec(dims, lambda i: (0,) * len(dims))
    return pl.pallas_call(
        _k1_body,
        grid=(T // BT1,),
        in_specs=[
            pl.BlockSpec((BT1, D), lambda i: (i, 0)),
            const(D, P),
            const(1, P),
            const(P, D),
            const(1, D),
        ],
        out_specs=[
            pl.BlockSpec((BT1, D), lambda i: (i, 0)),
            pl.BlockSpec((BT1, 128), lambda i: (i, 0)),
        ],
        out_shape=[
            jax.ShapeDtypeStruct((T, D), _F32),
            jax.ShapeDtypeStruct((T, 128), jnp.int32),
        ],
    )(x, pe_w.T, pe_b.reshape(1, P), pd_w, dec_bias.reshape(1, D))


def _k2a_body(pp_hbm, inv0_hbm, inv1_hbm, idx_hbm, bp_hbm, xch_hbm,
              ppv, hist_pub, histv, inv0v, inv1v, idxvf, bpv,
              runcnt, la, tots, pres, offs, blk0, nbv):
    sid = lax.axis_index("s")
    cid = lax.axis_index("c")
    tpt = T // 16
    lane = lax.iota(jnp.int32, 16)
    pltpu.sync_copy(pp_hbm.at[pl.ds(sid * tpt, tpt)], ppv)

    def zbody(p, _):
        runcnt[p] = 0
        return _

    lax.fori_loop(0, P, zbody, 0)

    def pass1(cc, _):
        for j in range(16):
            i = cc * 16 + j
            pk = ppv[i, pl.ds(0, 16)][0]
            p1 = lax.shift_right_logical(pk, 6)
            p2 = jnp.bitwise_and(pk, 63)
            r1 = runcnt[p1]
            la[2 * i] = p1 * 4096 + r1
            runcnt[p1] = r1 + 1
            r2 = runcnt[p2]
            la[2 * i + 1] = p2 * 4096 + r2
            runcnt[p2] = r2 + 1
        return _

    lax.fori_loop(0, 16, pass1, 0)

    for j in range(P // 16):
        acc = jnp.zeros((16,), jnp.int32)
        for l in range(16):
            acc = jnp.where(lane == l, runcnt[16 * j + l], acc)
        hist_pub[pl.ds(16 * j, 16)] = acc
    pltpu.sync_copy(hist_pub, xch_hbm.at[cid * 16 + sid])
    plsc.subcore_barrier()
    pltpu.sync_copy(xch_hbm.at[pl.ds(cid * 16, 16)], histv)

    for j in range(P // 16):
        tot = jnp.zeros((16,), jnp.int32)
        pre = jnp.zeros((16,), jnp.int32)
        for w2 in range(16):
            v = histv[w2, pl.ds(16 * j, 16)]
            tot = tot + v
            pre = pre + v * jnp.where(w2 < sid, 1, 0)
        for l in range(16):
            tots[16 * j + l] = tot[l]
            pres[16 * j + l] = pre[l]

    def obody(p, carry):
        t = tots[p]
        nb = lax.shift_right_logical(t + (BLK - 1), 7)
        offs[p] = carry * BLK + pres[p]
        blk0[p] = carry
        nbv[p] = nb
        return carry + nb

    totblk = lax.fori_loop(0, P, obody, 0)

    for j in range(NBLOCKS // 16):
        bpv[pl.ds(16 * j, 16)] = jnp.zeros((16,), jnp.int32)

    def bpbody(p, _):
        e = blk0[p] + nbv[p]
        for j in range(NBLOCKS // 16):
            bvec = lane + 16 * j
            sl = pl.ds(16 * j, 16)
            bpv[sl] = bpv[sl] + jnp.where(bvec >= e, 1, 0)
        return _

    lax.fori_loop(0, P, bpbody, 0)
    for j in range(NBLOCKS // 16):
        bvec = lane + 16 * j
        sl = pl.ds(16 * j, 16)
        bpv[sl] = jnp.where(bvec < totblk, bpv[sl], -1)

    @pl.when((sid == 0) & (cid == 0))
    def _bp():
        pltpu.sync_copy(bpv, bp_hbm)

    def pass2(cc, _):
        d0 = jnp.zeros((16,), jnp.int32)
        d1 = jnp.zeros((16,), jnp.int32)
        for j in range(16):
            i = cc * 16 + j
            a0 = la[2 * i]
            p0 = lax.shift_right_logical(a0, 12)
            r0 = jnp.bitwise_and(a0, 4095)
            d0 = jnp.where(lane == j, offs[p0] + r0, d0)
            a1 = la[2 * i + 1]
            p1 = lax.shift_right_logical(a1, 12)
            r1 = jnp.bitwise_and(a1, 4095)
            d1 = jnp.where(lane == j, offs[p1] + r1, d1)
        base = cc * 16
        inv0v[pl.ds(base, 16)] = d0
        inv1v[pl.ds(base, 16)] = d1
        h = lax.shift_right_logical(cc, 3)
        chunk = jnp.bitwise_and(lax.shift_right_logical(cc, 1), 3)
        colb = jnp.bitwise_and(cc, 1) * 16
        fb = h * 256 + chunk * 32 + colb
        idxvf[pl.ds(fb, 16)] = d0
        idxvf[pl.ds(fb + 128, 16)] = d1
        return _

    lax.fori_loop(0, 16, pass2, 0)

    @pl.when(cid == 0)
    def _wr():
        pltpu.sync_copy(inv0v, inv0_hbm.at[pl.ds(sid * tpt, tpt)])
        pltpu.sync_copy(inv1v, inv1_hbm.at[pl.ds(sid * tpt, tpt)])
        pltpu.sync_copy(idxvf, idx_hbm.at[pl.ds(sid * 512, 512)])


def _k2a(pp):
    mesh = plsc.VectorSubcoreMesh(core_axis_name="c", subcore_axis_name="s")
    tpt = T // 16
    f = functools.partial(
        pl.kernel,
        out_type=[
            jax.ShapeDtypeStruct((T,), jnp.int32),
            jax.ShapeDtypeStruct((T,), jnp.int32),
            jax.ShapeDtypeStruct((32 * 256,), jnp.int32),
            jax.ShapeDtypeStruct((NBLOCKS,), jnp.int32),
            jax.ShapeDtypeStruct((32, P), jnp.int32),
        ],
        mesh=mesh,
        scratch_types=[
            pltpu.VMEM((tpt, 128), jnp.int32),
            pltpu.VMEM((P,), jnp.int32),
            pltpu.VMEM((16, P), jnp.int32),
            pltpu.VMEM((tpt,), jnp.int32),
            pltpu.VMEM((tpt,), jnp.int32),
            pltpu.VMEM((512,), jnp.int32),
            pltpu.VMEM((NBLOCKS,), jnp.int32),
            pltpu.SMEM((P,), jnp.int32),
            pltpu.SMEM((2 * tpt,), jnp.int32),
            pltpu.SMEM((P,), jnp.int32),
            pltpu.SMEM((P,), jnp.int32),
            pltpu.SMEM((P,), jnp.int32),
            pltpu.SMEM((P,), jnp.int32),
            pltpu.SMEM((P,), jnp.int32),
        ],
    )(_k2a_body)
    return f(pp)


def _k2b_body(x_hbm, idx_hbm, xs_hbm, idxflat, idxv2, xrow, sem):
    v = lax.axis_index("s") * 2 + lax.axis_index("c")
    pltpu.sync_copy(idx_hbm.at[pl.ds(v * 256, 256)], idxflat)
    for r in range(8):
        for h in range(2):
            idxv2[r, pl.ds(h * 16, 16)] = idxflat[pl.ds(r * 32 + h * 16, 16)]
    for c in range(4):
        pltpu.sync_copy(x_hbm.at[pl.ds(v * 128 + c * 32, 32)], xrow)
        pltpu.async_copy(xrow, xs_hbm.at[idxv2.at[c]], sem).wait()
        pltpu.async_copy(xrow, xs_hbm.at[idxv2.at[4 + c]], sem).wait()


def _k2b(x, idx):
    mesh = plsc.VectorSubcoreMesh(core_axis_name="c", subcore_axis_name="s")
    f = functools.partial(
        pl.kernel,
        out_type=[jax.ShapeDtypeStruct((NROWS, D), _F32)],
        mesh=mesh,
        scratch_types=[
            pltpu.VMEM((256,), jnp.int32),
            pltpu.VMEM((8, 32), jnp.int32),
            pltpu.VMEM((32, D), _F32),
            pltpu.SemaphoreType.DMA,
        ],
    )(_k2b_body)
    return f(x, idx)[0]


def _k3_body(bp_ref, xs_ref, dn0, ce0, cb0, cd0, up0,
             dn1, ce1, cb1, cd1, up1, out_ref):
    g = pl.program_id(0)

    @pl.when(bp_ref[2 * g] >= 0)
    def _():
        iota_c = jax.lax.broadcasted_iota(jnp.int32, (BLK, C), 1).astype(_F32)
        for h, (dn, ce, cb, cd, up) in enumerate(
                ((dn0, ce0, cb0, cd0, up0), (dn1, ce1, cb1, cd1, up1))):
            xb = xs_ref[h * BLK:(h + 1) * BLK, :]
            xsub = jnp.dot(xb, dn[0], preferred_element_type=_F32)
            cl = jnp.dot(xsub, ce[0], preferred_element_type=_F32) + cb[0]
            mask, _i1, _i2 = _top2(cl, iota_c)
            codes = jnp.where(mask, 1.0, 0.0)
            csub = jnp.dot(codes, cd[0], preferred_element_type=_F32)
            out_ref[h * BLK:(h + 1) * BLK, :] = jnp.dot(
                csub, up[0], preferred_element_type=_F32)


def _k3(bp, xs, down_t, ce_t, ce_b3, cd_t, up_t):
    nsteps = NBLOCKS // 2
    wm0 = lambda b, bp: (jnp.maximum(bp[2 * b], 0), 0, 0)
    wm1 = lambda b, bp: (jnp.maximum(bp[2 * b + 1], 0), 0, 0)
    wspecs0 = [pl.BlockSpec((1, D, S), wm0), pl.BlockSpec((1, S, C), wm0),
               pl.BlockSpec((1, 1, C), wm0), pl.BlockSpec((1, C, S), wm0),
               pl.BlockSpec((1, S, D), wm0)]
    wspecs1 = [pl.BlockSpec((1, D, S), wm1), pl.BlockSpec((1, S, C), wm1),
               pl.BlockSpec((1, 1, C), wm1), pl.BlockSpec((1, C, S), wm1),
               pl.BlockSpec((1, S, D), wm1)]
    grid_spec = pltpu.PrefetchScalarGridSpec(
        num_scalar_prefetch=1,
        grid=(nsteps,),
        in_specs=[
            pl.BlockSpec((2 * BLK, D),
                         lambda b, bp: (jnp.where(bp[2 * b] >= 0, b, 0), 0)),
        ] + wspecs0 + wspecs1,
        out_specs=pl.BlockSpec(
            (2 * BLK, D),
            lambda b, bp: (jnp.where(bp[2 * b] >= 0, b, nsteps), 0)),
    )
    w = (down_t, ce_t, ce_b3, cd_t, up_t)
    return pl.pallas_call(
        _k3_body,
        grid_spec=grid_spec,
        out_shape=jax.ShapeDtypeStruct((NROWS + 2 * BLK, D), _F32),
    )(bp, xs, *w, *w)


def _k4_body(rows_hbm, base_hbm, inv0_hbm, inv1_hbm, out_hbm,
             inv0v, inv1v, b0a, b0b, b1a, b1b, bba, bbb, s0, s1):
    v = lax.axis_index("s") * 2 + lax.axis_index("c")
    t0 = v * 128
    bufs0, bufs1, bbs, sems = [b0a, b0b], [b1a, b1b], [bba, bbb], [s0, s1]
    pltpu.sync_copy(inv0_hbm.at[pl.ds(t0, 128)], inv0v)
    pltpu.sync_copy(inv1_hbm.at[pl.ds(t0, 128)], inv1v)

    def start(c, slot):
        sl = pl.ds(c * 16, 16)
        return (
            pltpu.async_copy(rows_hbm.at[inv0v.at[sl]], bufs0[slot],
                             sems[slot]),
            pltpu.async_copy(rows_hbm.at[inv1v.at[sl]], bufs1[slot],
                             sems[slot]),
            pltpu.async_copy(base_hbm.at[pl.ds(t0 + c * 16, 16)], bbs[slot],
                             sems[slot]),
        )

    pend = start(0, 0)
    for c in range(8):
        slot = c & 1
        if c < 7:
            nxt = start(c + 1, 1 - slot)
        for dsc in pend:
            dsc.wait()
        bb, buf0, buf1 = bbs[slot], bufs0[slot], bufs1[slot]

        def radd(r, _):
            for h in range(D // 16):
                sl = pl.ds(h * 16, 16)
                bb[r, sl] = bb[r, sl] + buf0[r, sl] + buf1[r, sl]
            return _

        lax.fori_loop(0, 16, radd, 0)
        pltpu.sync_copy(bb, out_hbm.at[pl.ds(t0 + c * 16, 16)])
        if c < 7:
            pend = nxt


def _k4(rows, base, inv0, inv1):
    mesh = plsc.VectorSubcoreMesh(core_axis_name="c", subcore_axis_name="s")
    f = functools.partial(
        pl.kernel,
        out_type=[jax.ShapeDtypeStruct((T, D), _F32)],
        mesh=mesh,
        scratch_types=[
            pltpu.VMEM((128,), jnp.int32),
            pltpu.VMEM((128,), jnp.int32),
            pltpu.VMEM((16, D), _F32),
            pltpu.VMEM((16, D), _F32),
            pltpu.VMEM((16, D), _F32),
            pltpu.VMEM((16, D), _F32),
            pltpu.VMEM((16, D), _F32),
            pltpu.VMEM((16, D), _F32),
            pltpu.SemaphoreType.DMA,
            pltpu.SemaphoreType.DMA,
        ],
    )(_k4_body)
    return f(rows, base, inv0, inv1)[0]


def kernel(x, pe_w, pe_b, pd_w, down_w, up_w, ce_w, ce_b, cd_w, dec_bias):
    down_t = jnp.transpose(down_w, (0, 2, 1))
    ce_t = jnp.transpose(ce_w, (0, 2, 1))
    ce_b3 = ce_b.reshape(P, 1, C)
    cd_t = jnp.transpose(cd_w, (0, 2, 1))
    up_t = jnp.transpose(up_w, (0, 2, 1))

    base, pp = _k1(x, pe_w, pe_b, pd_w, dec_bias)
    inv0, inv1, idx, bp, _ = _k2a(pp)
    xs = _k2b(x, idx)
    rows = _k3(bp, xs, down_t, ce_t, ce_b3, cd_t, up_t)
    return _k4(rows, base, inv0, inv1)


# speedup vs baseline: 27.8367x; 1.0143x over previous
import functools

import jax
import jax.numpy as jnp
from jax import lax
from jax.experimental import pallas as pl
from jax.experimental.pallas import tpu as pltpu
from jax.experimental.pallas import tpu_sc as plsc

D = 768
P = 64
S = 32
C = 128
T = 4096
BLK = 128
NBLOCKS = 128
NROWS = NBLOCKS * BLK
BT1 = 512
KPACK = 4

_NEG = -3.0e38
_F32 = jnp.float32


def _top2(logits, iota_f):
    big = 1.0e9
    m1 = jnp.max(logits, axis=1, keepdims=True)
    i1 = jnp.min(jnp.where(logits == m1, iota_f, big), axis=1, keepdims=True)
    sel1 = iota_f == i1
    rest = jnp.where(sel1, _NEG, logits)
    m2 = jnp.max(rest, axis=1, keepdims=True)
    i2 = jnp.min(jnp.where(rest == m2, iota_f, big), axis=1, keepdims=True)
    return jnp.logical_or(sel1, iota_f == i2), i1, i2


def _k1_body(x_ref, pewt_ref, peb_ref, pdw_ref, bias_ref, base_ref, pp_ref):
    xb = x_ref[:]
    iota_p = jax.lax.broadcasted_iota(jnp.int32, (BT1, P), 1).astype(_F32)
    logits = jnp.dot(xb, pewt_ref[:], preferred_element_type=_F32) + peb_ref[:]
    mask, i1, i2 = _top2(logits, iota_p)
    pcodes = jnp.where(mask, 1.0, 0.0)
    base_ref[:] = jnp.dot(pcodes, pdw_ref[:],
                          preferred_element_type=_F32) + bias_ref[:]
    packed = i1 * 64.0 + i2
    pp_ref[:] = jnp.broadcast_to(packed, (BT1, 128)).astype(jnp.int32)


def _k1(x, pe_w, pe_b, pd_w, dec_bias):
    const = lambda *dims: pl.BlockSpec(dims, lambda i: (0,) * len(dims))
    return pl.pallas_call(
        _k1_body,
        grid=(T // BT1,),
        in_specs=[
            pl.BlockSpec((BT1, D), lambda i: (i, 0)),
            const(D, P),
            const(1, P),
            const(P, D),
            const(1, D),
        ],
        out_specs=[
            pl.BlockSpec((BT1, D), lambda i: (i, 0)),
            pl.BlockSpec((BT1, 128), lambda i: (i, 0)),
        ],
        out_shape=[
            jax.ShapeDtypeStruct((T, D), _F32),
            jax.ShapeDtypeStruct((T, 128), jnp.int32),
        ],
    )(x, pe_w.T, pe_b.reshape(1, P), pd_w, dec_bias.reshape(1, D))


def _k2a_body(pp_hbm, inv0_hbm, inv1_hbm, idx_hbm, bp_hbm, xch_hbm,
              ppv, hist_pub, histv, inv0v, inv1v, idxvf, bpv,
              runcnt, la, tots, pres, offs, blk0, nbv):
    sid = lax.axis_index("s")
    cid = lax.axis_index("c")
    tpt = T // 16
    lane = lax.iota(jnp.int32, 16)
    pltpu.sync_copy(pp_hbm.at[pl.ds(sid * tpt, tpt)], ppv)

    def zbody(p, _):
        runcnt[p] = 0
        return _

    lax.fori_loop(0, P, zbody, 0)

    def pass1(cc, _):
        for j in range(16):
            i = cc * 16 + j
            pk = ppv[i, pl.ds(0, 16)][0]
            p1 = lax.shift_right_logical(pk, 6)
            p2 = jnp.bitwise_and(pk, 63)
            r1 = runcnt[p1]
            la[2 * i] = p1 * 4096 + r1
            runcnt[p1] = r1 + 1
            r2 = runcnt[p2]
            la[2 * i + 1] = p2 * 4096 + r2
            runcnt[p2] = r2 + 1
        return _

    lax.fori_loop(0, 16, pass1, 0)

    for j in range(P // 16):
        acc = jnp.zeros((16,), jnp.int32)
        for l in range(16):
            acc = jnp.where(lane == l, runcnt[16 * j + l], acc)
        hist_pub[pl.ds(16 * j, 16)] = acc
    pltpu.sync_copy(hist_pub, xch_hbm.at[cid * 16 + sid])
    plsc.subcore_barrier()
    pltpu.sync_copy(xch_hbm.at[pl.ds(cid * 16, 16)], histv)

    for j in range(P // 16):
        tot = jnp.zeros((16,), jnp.int32)
        pre = jnp.zeros((16,), jnp.int32)
        for w2 in range(16):
            v = histv[w2, pl.ds(16 * j, 16)]
            tot = tot + v
            pre = pre + v * jnp.where(w2 < sid, 1, 0)
        for l in range(16):
            tots[16 * j + l] = tot[l]
            pres[16 * j + l] = pre[l]

    def obody(p, carry):
        t = tots[p]
        nb = lax.shift_right_logical(t + (BLK - 1), 7)
        offs[p] = carry * BLK + pres[p]
        blk0[p] = carry
        nbv[p] = nb
        return carry + nb

    totblk = lax.fori_loop(0, P, obody, 0)

    for j in range(NBLOCKS // 16):
        bpv[pl.ds(16 * j, 16)] = jnp.zeros((16,), jnp.int32)

    def bpbody(p, _):
        e = blk0[p] + nbv[p]
        for j in range(NBLOCKS // 16):
            bvec = lane + 16 * j
            sl = pl.ds(16 * j, 16)
            bpv[sl] = bpv[sl] + jnp.where(bvec >= e, 1, 0)
        return _

    lax.fori_loop(0, P, bpbody, 0)
    for j in range(NBLOCKS // 16):
        bvec = lane + 16 * j
        sl = pl.ds(16 * j, 16)
        bpv[sl] = jnp.where(bvec < totblk, bpv[sl], -1)

    @pl.when((sid == 0) & (cid == 0))
    def _bp():
        pltpu.sync_copy(bpv, bp_hbm)

    def pass2(cc, _):
        d0 = jnp.zeros((16,), jnp.int32)
        d1 = jnp.zeros((16,), jnp.int32)
        for j in range(16):
            i = cc * 16 + j
            a0 = la[2 * i]
            p0 = lax.shift_right_logical(a0, 12)
            r0 = jnp.bitwise_and(a0, 4095)
            d0 = jnp.where(lane == j, offs[p0] + r0, d0)
            a1 = la[2 * i + 1]
            p1 = lax.shift_right_logical(a1, 12)
            r1 = jnp.bitwise_and(a1, 4095)
            d1 = jnp.where(lane == j, offs[p1] + r1, d1)
        base = cc * 16
        inv0v[pl.ds(base, 16)] = d0
        inv1v[pl.ds(base, 16)] = d1
        h = lax.shift_right_logical(cc, 3)
        chunk = jnp.bitwise_and(lax.shift_right_logical(cc, 1), 3)
        colb = jnp.bitwise_and(cc, 1) * 16
        fb = h * 256 + chunk * 32 + colb
        idxvf[pl.ds(fb, 16)] = d0
        idxvf[pl.ds(fb + 128, 16)] = d1
        return _

    lax.fori_loop(0, 16, pass2, 0)

    @pl.when(cid == 0)
    def _wr():
        pltpu.sync_copy(inv0v, inv0_hbm.at[pl.ds(sid * tpt, tpt)])
        pltpu.sync_copy(inv1v, inv1_hbm.at[pl.ds(sid * tpt, tpt)])
        pltpu.sync_copy(idxvf, idx_hbm.at[pl.ds(sid * 512, 512)])


def _k2a(pp):
    mesh = plsc.VectorSubcoreMesh(core_axis_name="c", subcore_axis_name="s")
    tpt = T // 16
    f = functools.partial(
        pl.kernel,
        out_type=[
            jax.ShapeDtypeStruct((T,), jnp.int32),
            jax.ShapeDtypeStruct((T,), jnp.int32),
            jax.ShapeDtypeStruct((32 * 256,), jnp.int32),
            jax.ShapeDtypeStruct((NBLOCKS,), jnp.int32),
            jax.ShapeDtypeStruct((32, P), jnp.int32),
        ],
        mesh=mesh,
        scratch_types=[
            pltpu.VMEM((tpt, 128), jnp.int32),
            pltpu.VMEM((P,), jnp.int32),
            pltpu.VMEM((16, P), jnp.int32),
            pltpu.VMEM((tpt,), jnp.int32),
            pltpu.VMEM((tpt,), jnp.int32),
            pltpu.VMEM((512,), jnp.int32),
            pltpu.VMEM((NBLOCKS,), jnp.int32),
            pltpu.SMEM((P,), jnp.int32),
            pltpu.SMEM((2 * tpt,), jnp.int32),
            pltpu.SMEM((P,), jnp.int32),
            pltpu.SMEM((P,), jnp.int32),
            pltpu.SMEM((P,), jnp.int32),
            pltpu.SMEM((P,), jnp.int32),
            pltpu.SMEM((P,), jnp.int32),
        ],
    )(_k2a_body)
    return f(pp)


def _k2b_body(x_hbm, idx_hbm, xs_hbm, idxflat, idxv2, xrow, sem):
    v = lax.axis_index("s") * 2 + lax.axis_index("c")
    pltpu.sync_copy(idx_hbm.at[pl.ds(v * 256, 256)], idxflat)
    for r in range(8):
        for h in range(2):
            idxv2[r, pl.ds(h * 16, 16)] = idxflat[pl.ds(r * 32 + h * 16, 16)]
    for c in range(4):
        pltpu.sync_copy(x_hbm.at[pl.ds(v * 128 + c * 32, 32)], xrow)
        pltpu.async_copy(xrow, xs_hbm.at[idxv2.at[c]], sem).wait()
        pltpu.async_copy(xrow, xs_hbm.at[idxv2.at[4 + c]], sem).wait()


def _k2b(x, idx):
    mesh = plsc.VectorSubcoreMesh(core_axis_name="c", subcore_axis_name="s")
    f = functools.partial(
        pl.kernel,
        out_type=[jax.ShapeDtypeStruct((NROWS, D), _F32)],
        mesh=mesh,
        scratch_types=[
            pltpu.VMEM((256,), jnp.int32),
            pltpu.VMEM((8, 32), jnp.int32),
            pltpu.VMEM((32, D), _F32),
            pltpu.SemaphoreType.DMA,
        ],
    )(_k2b_body)
    return f(x, idx)[0]


def _k3_body(bp_ref, xs_ref, *ws_and_out):
    ws, out_ref = ws_and_out[:-1], ws_and_out[-1]
    g = pl.program_id(0)

    @pl.when(bp_ref[KPACK * g] >= 0)
    def _():
        iota_c = jax.lax.broadcasted_iota(jnp.int32, (BLK, C), 1).astype(_F32)
        for h in range(KPACK):
            dn, ce, cb, cd, up = ws[5 * h:5 * h + 5]
            xb = xs_ref[h * BLK:(h + 1) * BLK, :]
            xsub = jnp.dot(xb, dn[0], preferred_element_type=_F32)
            cl = jnp.dot(xsub, ce[0], preferred_element_type=_F32) + cb[0]
            mask, _i1, _i2 = _top2(cl, iota_c)
            codes = jnp.where(mask, 1.0, 0.0)
            csub = jnp.dot(codes, cd[0], preferred_element_type=_F32)
            out_ref[h * BLK:(h + 1) * BLK, :] = jnp.dot(
                csub, up[0], preferred_element_type=_F32)


def _k3(bp, xs, down_t, ce_t, ce_b3, cd_t, up_t):
    nsteps = NBLOCKS // KPACK
    wspecs = []
    for h in range(KPACK):
        wm = (lambda hh: lambda b, bp: (jnp.maximum(bp[KPACK * b + hh], 0),
                                        0, 0))(h)
        wspecs += [pl.BlockSpec((1, D, S), wm), pl.BlockSpec((1, S, C), wm),
                   pl.BlockSpec((1, 1, C), wm), pl.BlockSpec((1, C, S), wm),
                   pl.BlockSpec((1, S, D), wm)]
    grid_spec = pltpu.PrefetchScalarGridSpec(
        num_scalar_prefetch=1,
        grid=(nsteps,),
        in_specs=[
            pl.BlockSpec((KPACK * BLK, D),
                         lambda b, bp: (jnp.where(bp[KPACK * b] >= 0, b, 0),
                                        0)),
        ] + wspecs,
        out_specs=pl.BlockSpec(
            (KPACK * BLK, D),
            lambda b, bp: (jnp.where(bp[KPACK * b] >= 0, b, nsteps), 0)),
    )
    w = (down_t, ce_t, ce_b3, cd_t, up_t)
    return pl.pallas_call(
        _k3_body,
        grid_spec=grid_spec,
        out_shape=jax.ShapeDtypeStruct((NROWS + KPACK * BLK, D), _F32),
    )(bp, xs, *(w * KPACK))


def _k4_body(rows_hbm, base_hbm, inv0_hbm, inv1_hbm, out_hbm,
             inv0v, inv1v, b0a, b0b, b1a, b1b, bba, bbb, s0, s1):
    v = lax.axis_index("s") * 2 + lax.axis_index("c")
    t0 = v * 128
    bufs0, bufs1, bbs, sems = [b0a, b0b], [b1a, b1b], [bba, bbb], [s0, s1]
    pltpu.sync_copy(inv0_hbm.at[pl.ds(t0, 128)], inv0v)
    pltpu.sync_copy(inv1_hbm.at[pl.ds(t0, 128)], inv1v)

    def start(c, slot):
        sl = pl.ds(c * 16, 16)
        return (
            pltpu.async_copy(rows_hbm.at[inv0v.at[sl]], bufs0[slot],
                             sems[slot]),
            pltpu.async_copy(rows_hbm.at[inv1v.at[sl]], bufs1[slot],
                             sems[slot]),
            pltpu.async_copy(base_hbm.at[pl.ds(t0 + c * 16, 16)], bbs[slot],
                             sems[slot]),
        )

    pend = start(0, 0)
    for c in range(8):
        slot = c & 1
        if c < 7:
            nxt = start(c + 1, 1 - slot)
        for dsc in pend:
            dsc.wait()
        bb, buf0, buf1 = bbs[slot], bufs0[slot], bufs1[slot]

        def radd(r, _):
            for h in range(D // 16):
                sl = pl.ds(h * 16, 16)
                bb[r, sl] = bb[r, sl] + buf0[r, sl] + buf1[r, sl]
            return _

        lax.fori_loop(0, 16, radd, 0)
        pltpu.sync_copy(bb, out_hbm.at[pl.ds(t0 + c * 16, 16)])
        if c < 7:
            pend = nxt


def _k4(rows, base, inv0, inv1):
    mesh = plsc.VectorSubcoreMesh(core_axis_name="c", subcore_axis_name="s")
    f = functools.partial(
        pl.kernel,
        out_type=[jax.ShapeDtypeStruct((T, D), _F32)],
        mesh=mesh,
        scratch_types=[
            pltpu.VMEM((128,), jnp.int32),
            pltpu.VMEM((128,), jnp.int32),
            pltpu.VMEM((16, D), _F32),
            pltpu.VMEM((16, D), _F32),
            pltpu.VMEM((16, D), _F32),
            pltpu.VMEM((16, D), _F32),
            pltpu.VMEM((16, D), _F32),
            pltpu.VMEM((16, D), _F32),
            pltpu.SemaphoreType.DMA,
            pltpu.SemaphoreType.DMA,
        ],
    )(_k4_body)
    return f(rows, base, inv0, inv1)[0]


def kernel(x, pe_w, pe_b, pd_w, down_w, up_w, ce_w, ce_b, cd_w, dec_bias):
    down_t = jnp.transpose(down_w, (0, 2, 1))
    ce_t = jnp.transpose(ce_w, (0, 2, 1))
    ce_b3 = ce_b.reshape(P, 1, C)
    cd_t = jnp.transpose(cd_w, (0, 2, 1))
    up_t = jnp.transpose(up_w, (0, 2, 1))

    base, pp = _k1(x, pe_w, pe_b, pd_w, dec_bias)
    inv0, inv1, idx, bp, _ = _k2a(pp)
    xs = _k2b(x, idx)
    rows = _k3(bp, xs, down_t, ce_t, ce_b3, cd_t, up_t)
    return _k4(rows, base, inv0, inv1)


# speedup vs baseline: 27.8397x; 1.0001x over previous
import functools

import jax
import jax.numpy as jnp
from jax import lax
from jax.experimental import pallas as pl
from jax.experimental.pallas import tpu as pltpu
from jax.experimental.pallas import tpu_sc as plsc

D = 768
P = 64
S = 32
C = 128
T = 4096
BLK = 128
NBLOCKS = 128
NROWS = NBLOCKS * BLK
BT1 = 512
KPACK = 4

_NEG = -3.0e38
_F32 = jnp.float32


def _top2(logits, iota_f):
    big = 1.0e9
    m1 = jnp.max(logits, axis=1, keepdims=True)
    i1 = jnp.min(jnp.where(logits == m1, iota_f, big), axis=1, keepdims=True)
    sel1 = iota_f == i1
    rest = jnp.where(sel1, _NEG, logits)
    m2 = jnp.max(rest, axis=1, keepdims=True)
    i2 = jnp.min(jnp.where(rest == m2, iota_f, big), axis=1, keepdims=True)
    return jnp.logical_or(sel1, iota_f == i2), i1, i2


def _k1_body(x_ref, pewt_ref, peb_ref, pdw_ref, bias_ref, base_ref, pp_ref):
    xb = x_ref[:]
    iota_p = jax.lax.broadcasted_iota(jnp.int32, (BT1, P), 1).astype(_F32)
    logits = jnp.dot(xb, pewt_ref[:], preferred_element_type=_F32) + peb_ref[:]
    mask, i1, i2 = _top2(logits, iota_p)
    pcodes = jnp.where(mask, 1.0, 0.0)
    base_ref[:] = jnp.dot(pcodes, pdw_ref[:],
                          preferred_element_type=_F32) + bias_ref[:]
    packed = i1 * 64.0 + i2
    pp_ref[:] = jnp.broadcast_to(packed, (BT1, 16)).astype(jnp.int32)


def _k1(x, pe_w, pe_b, pd_w, dec_bias):
    const = lambda *dims: pl.BlockSpec(dims, lambda i: (0,) * len(dims))
    return pl.pallas_call(
        _k1_body,
        grid=(T // BT1,),
        in_specs=[
            pl.BlockSpec((BT1, D), lambda i: (i, 0)),
            const(D, P),
            const(1, P),
            const(P, D),
            const(1, D),
        ],
        out_specs=[
            pl.BlockSpec((BT1, D), lambda i: (i, 0)),
            pl.BlockSpec((BT1, 16), lambda i: (i, 0)),
        ],
        out_shape=[
            jax.ShapeDtypeStruct((T, D), _F32),
            jax.ShapeDtypeStruct((T, 16), jnp.int32),
        ],
    )(x, pe_w.T, pe_b.reshape(1, P), pd_w, dec_bias.reshape(1, D))


def _k2a_body(pp_hbm, inv0_hbm, inv1_hbm, idx_hbm, bp_hbm, xch_hbm,
              ppv, hist_pub, histv, inv0v, inv1v, idxvf, bpv,
              runcnt, la, tots, pres, offs, blk0, nbv):
    sid = lax.axis_index("s")
    cid = lax.axis_index("c")
    tpt = T // 16
    lane = lax.iota(jnp.int32, 16)
    pltpu.sync_copy(pp_hbm.at[pl.ds(sid * tpt, tpt)], ppv)

    def zbody(p, _):
        runcnt[p] = 0
        return _

    lax.fori_loop(0, P, zbody, 0)

    def pass1(cc, _):
        for j in range(16):
            i = cc * 16 + j
            pk = ppv[i, pl.ds(0, 16)][0]
            p1 = lax.shift_right_logical(pk, 6)
            p2 = jnp.bitwise_and(pk, 63)
            r1 = runcnt[p1]
            la[2 * i] = p1 * 4096 + r1
            runcnt[p1] = r1 + 1
            r2 = runcnt[p2]
            la[2 * i + 1] = p2 * 4096 + r2
            runcnt[p2] = r2 + 1
        return _

    lax.fori_loop(0, 16, pass1, 0)

    for j in range(P // 16):
        acc = jnp.zeros((16,), jnp.int32)
        for l in range(16):
            acc = jnp.where(lane == l, runcnt[16 * j + l], acc)
        hist_pub[pl.ds(16 * j, 16)] = acc
    pltpu.sync_copy(hist_pub, xch_hbm.at[cid * 16 + sid])
    plsc.subcore_barrier()
    pltpu.sync_copy(xch_hbm.at[pl.ds(cid * 16, 16)], histv)

    for j in range(P // 16):
        tot = jnp.zeros((16,), jnp.int32)
        pre = jnp.zeros((16,), jnp.int32)
        for w2 in range(16):
            v = histv[w2, pl.ds(16 * j, 16)]
            tot = tot + v
            pre = pre + v * jnp.where(w2 < sid, 1, 0)
        for l in range(16):
            tots[16 * j + l] = tot[l]
            pres[16 * j + l] = pre[l]

    def obody(p, carry):
        t = tots[p]
        nb = lax.shift_right_logical(t + (BLK - 1), 7)
        offs[p] = carry * BLK + pres[p]
        blk0[p] = carry
        nbv[p] = nb
        return carry + nb

    totblk = lax.fori_loop(0, P, obody, 0)

    for j in range(NBLOCKS // 16):
        bpv[pl.ds(16 * j, 16)] = jnp.zeros((16,), jnp.int32)

    def bpbody(p, _):
        e = blk0[p] + nbv[p]
        for j in range(NBLOCKS // 16):
            bvec = lane + 16 * j
            sl = pl.ds(16 * j, 16)
            bpv[sl] = bpv[sl] + jnp.where(bvec >= e, 1, 0)
        return _

    lax.fori_loop(0, P, bpbody, 0)
    for j in range(NBLOCKS // 16):
        bvec = lane + 16 * j
        sl = pl.ds(16 * j, 16)
        bpv[sl] = jnp.where(bvec < totblk, bpv[sl], -1)

    @pl.when((sid == 0) & (cid == 0))
    def _bp():
        pltpu.sync_copy(bpv, bp_hbm)

    def pass2(cc, _):
        d0 = jnp.zeros((16,), jnp.int32)
        d1 = jnp.zeros((16,), jnp.int32)
        for j in range(16):
            i = cc * 16 + j
            a0 = la[2 * i]
            p0 = lax.shift_right_logical(a0, 12)
            r0 = jnp.bitwise_and(a0, 4095)
            d0 = jnp.where(lane == j, offs[p0] + r0, d0)
            a1 = la[2 * i + 1]
            p1 = lax.shift_right_logical(a1, 12)
            r1 = jnp.bitwise_and(a1, 4095)
            d1 = jnp.where(lane == j, offs[p1] + r1, d1)
        base = cc * 16
        inv0v[pl.ds(base, 16)] = d0
        inv1v[pl.ds(base, 16)] = d1
        h = lax.shift_right_logical(cc, 3)
        chunk = jnp.bitwise_and(lax.shift_right_logical(cc, 1), 3)
        colb = jnp.bitwise_and(cc, 1) * 16
        fb = h * 256 + chunk * 32 + colb
        idxvf[pl.ds(fb, 16)] = d0
        idxvf[pl.ds(fb + 128, 16)] = d1
        return _

    lax.fori_loop(0, 16, pass2, 0)

    @pl.when(cid == 0)
    def _wr():
        pltpu.sync_copy(inv0v, inv0_hbm.at[pl.ds(sid * tpt, tpt)])
        pltpu.sync_copy(inv1v, inv1_hbm.at[pl.ds(sid * tpt, tpt)])
        pltpu.sync_copy(idxvf, idx_hbm.at[pl.ds(sid * 512, 512)])


def _k2a(pp):
    mesh = plsc.VectorSubcoreMesh(core_axis_name="c", subcore_axis_name="s")
    tpt = T // 16
    f = functools.partial(
        pl.kernel,
        out_type=[
            jax.ShapeDtypeStruct((T,), jnp.int32),
            jax.ShapeDtypeStruct((T,), jnp.int32),
            jax.ShapeDtypeStruct((32 * 256,), jnp.int32),
            jax.ShapeDtypeStruct((NBLOCKS,), jnp.int32),
            jax.ShapeDtypeStruct((32, P), jnp.int32),
        ],
        mesh=mesh,
        scratch_types=[
            pltpu.VMEM((tpt, 16), jnp.int32),
            pltpu.VMEM((P,), jnp.int32),
            pltpu.VMEM((16, P), jnp.int32),
            pltpu.VMEM((tpt,), jnp.int32),
            pltpu.VMEM((tpt,), jnp.int32),
            pltpu.VMEM((512,), jnp.int32),
            pltpu.VMEM((NBLOCKS,), jnp.int32),
            pltpu.SMEM((P,), jnp.int32),
            pltpu.SMEM((2 * tpt,), jnp.int32),
            pltpu.SMEM((P,), jnp.int32),
            pltpu.SMEM((P,), jnp.int32),
            pltpu.SMEM((P,), jnp.int32),
            pltpu.SMEM((P,), jnp.int32),
            pltpu.SMEM((P,), jnp.int32),
        ],
    )(_k2a_body)
    return f(pp)


def _k2b_body(x_hbm, idx_hbm, xs_hbm, idxflat, idxv2, xrow, sem):
    v = lax.axis_index("s") * 2 + lax.axis_index("c")
    pltpu.sync_copy(idx_hbm.at[pl.ds(v * 256, 256)], idxflat)
    for r in range(8):
        for h in range(2):
            idxv2[r, pl.ds(h * 16, 16)] = idxflat[pl.ds(r * 32 + h * 16, 16)]
    for c in range(4):
        pltpu.sync_copy(x_hbm.at[pl.ds(v * 128 + c * 32, 32)], xrow)
        pltpu.async_copy(xrow, xs_hbm.at[idxv2.at[c]], sem).wait()
        pltpu.async_copy(xrow, xs_hbm.at[idxv2.at[4 + c]], sem).wait()


def _k2b(x, idx):
    mesh = plsc.VectorSubcoreMesh(core_axis_name="c", subcore_axis_name="s")
    f = functools.partial(
        pl.kernel,
        out_type=[jax.ShapeDtypeStruct((NROWS, D), _F32)],
        mesh=mesh,
        scratch_types=[
            pltpu.VMEM((256,), jnp.int32),
            pltpu.VMEM((8, 32), jnp.int32),
            pltpu.VMEM((32, D), _F32),
            pltpu.SemaphoreType.DMA,
        ],
    )(_k2b_body)
    return f(x, idx)[0]


def _k3_body(bp_ref, xs_ref, *ws_and_out):
    ws, out_ref = ws_and_out[:-1], ws_and_out[-1]
    g = pl.program_id(0)

    @pl.when(bp_ref[KPACK * g] >= 0)
    def _():
        iota_c = jax.lax.broadcasted_iota(jnp.int32, (BLK, C), 1).astype(_F32)
        for h in range(KPACK):
            dn, ce, cb, cd, up = ws[5 * h:5 * h + 5]
            xb = xs_ref[h * BLK:(h + 1) * BLK, :]
            xsub = jnp.dot(xb, dn[0], preferred_element_type=_F32)
            cl = jnp.dot(xsub, ce[0], preferred_element_type=_F32) + cb[0]
            mask, _i1, _i2 = _top2(cl, iota_c)
            codes = jnp.where(mask, 1.0, 0.0)
            csub = jnp.dot(codes, cd[0], preferred_element_type=_F32)
            out_ref[h * BLK:(h + 1) * BLK, :] = jnp.dot(
                csub, up[0], preferred_element_type=_F32)


def _k3(bp, xs, down_t, ce_t, ce_b3, cd_t, up_t):
    nsteps = NBLOCKS // KPACK
    wspecs = []
    for h in range(KPACK):
        wm = (lambda hh: lambda b, bp: (jnp.maximum(bp[KPACK * b + hh], 0),
                                        0, 0))(h)
        wspecs += [pl.BlockSpec((1, D, S), wm), pl.BlockSpec((1, S, C), wm),
                   pl.BlockSpec((1, 1, C), wm), pl.BlockSpec((1, C, S), wm),
                   pl.BlockSpec((1, S, D), wm)]
    grid_spec = pltpu.PrefetchScalarGridSpec(
        num_scalar_prefetch=1,
        grid=(nsteps,),
        in_specs=[
            pl.BlockSpec((KPACK * BLK, D),
                         lambda b, bp: (jnp.where(bp[KPACK * b] >= 0, b, 0),
                                        0)),
        ] + wspecs,
        out_specs=pl.BlockSpec(
            (KPACK * BLK, D),
            lambda b, bp: (jnp.where(bp[KPACK * b] >= 0, b, nsteps), 0)),
    )
    w = (down_t, ce_t, ce_b3, cd_t, up_t)
    return pl.pallas_call(
        _k3_body,
        grid_spec=grid_spec,
        out_shape=jax.ShapeDtypeStruct((NROWS + KPACK * BLK, D), _F32),
    )(bp, xs, *(w * KPACK))


def _k4_body(rows_hbm, base_hbm, inv0_hbm, inv1_hbm, out_hbm,
             inv0v, inv1v, b0a, b0b, b1a, b1b, bba, bbb, s0, s1):
    v = lax.axis_index("s") * 2 + lax.axis_index("c")
    t0 = v * 128
    bufs0, bufs1, bbs, sems = [b0a, b0b], [b1a, b1b], [bba, bbb], [s0, s1]
    pltpu.sync_copy(inv0_hbm.at[pl.ds(t0, 128)], inv0v)
    pltpu.sync_copy(inv1_hbm.at[pl.ds(t0, 128)], inv1v)

    def start(c, slot):
        sl = pl.ds(c * 16, 16)
        return (
            pltpu.async_copy(rows_hbm.at[inv0v.at[sl]], bufs0[slot],
                             sems[slot]),
            pltpu.async_copy(rows_hbm.at[inv1v.at[sl]], bufs1[slot],
                             sems[slot]),
            pltpu.async_copy(base_hbm.at[pl.ds(t0 + c * 16, 16)], bbs[slot],
                             sems[slot]),
        )

    pend = start(0, 0)
    for c in range(8):
        slot = c & 1
        if c < 7:
            nxt = start(c + 1, 1 - slot)
        for dsc in pend:
            dsc.wait()
        bb, buf0, buf1 = bbs[slot], bufs0[slot], bufs1[slot]

        def radd(r, _):
            for h in range(D // 16):
                sl = pl.ds(h * 16, 16)
                bb[r, sl] = bb[r, sl] + buf0[r, sl] + buf1[r, sl]
            return _

        lax.fori_loop(0, 16, radd, 0)
        pltpu.sync_copy(bb, out_hbm.at[pl.ds(t0 + c * 16, 16)])
        if c < 7:
            pend = nxt


def _k4(rows, base, inv0, inv1):
    mesh = plsc.VectorSubcoreMesh(core_axis_name="c", subcore_axis_name="s")
    f = functools.partial(
        pl.kernel,
        out_type=[jax.ShapeDtypeStruct((T, D), _F32)],
        mesh=mesh,
        scratch_types=[
            pltpu.VMEM((128,), jnp.int32),
            pltpu.VMEM((128,), jnp.int32),
            pltpu.VMEM((16, D), _F32),
            pltpu.VMEM((16, D), _F32),
            pltpu.VMEM((16, D), _F32),
            pltpu.VMEM((16, D), _F32),
            pltpu.VMEM((16, D), _F32),
            pltpu.VMEM((16, D), _F32),
            pltpu.SemaphoreType.DMA,
            pltpu.SemaphoreType.DMA,
        ],
    )(_k4_body)
    return f(rows, base, inv0, inv1)[0]


def kernel(x, pe_w, pe_b, pd_w, down_w, up_w, ce_w, ce_b, cd_w, dec_bias):
    down_t = jnp.transpose(down_w, (0, 2, 1))
    ce_t = jnp.transpose(ce_w, (0, 2, 1))
    ce_b3 = ce_b.reshape(P, 1, C)
    cd_t = jnp.transpose(cd_w, (0, 2, 1))
    up_t = jnp.transpose(up_w, (0, 2, 1))

    base, pp = _k1(x, pe_w, pe_b, pd_w, dec_bias)
    inv0, inv1, idx, bp, _ = _k2a(pp)
    xs = _k2b(x, idx)
    rows = _k3(bp, xs, down_t, ce_t, ce_b3, cd_t, up_t)
    return _k4(rows, base, inv0, inv1)


# speedup vs baseline: 28.0239x; 1.0066x over previous
import functools

import jax
import jax.numpy as jnp
from jax import lax
from jax.experimental import pallas as pl
from jax.experimental.pallas import tpu as pltpu
from jax.experimental.pallas import tpu_sc as plsc

D = 768
P = 64
S = 32
C = 128
T = 4096
BLK = 128
NBLOCKS = 128
NROWS = NBLOCKS * BLK
BT1 = 512
KPACK = 4

_NEG = -3.0e38
_F32 = jnp.float32


def _top2(logits, iota_f):
    big = 1.0e9
    m1 = jnp.max(logits, axis=1, keepdims=True)
    i1 = jnp.min(jnp.where(logits == m1, iota_f, big), axis=1, keepdims=True)
    sel1 = iota_f == i1
    rest = jnp.where(sel1, _NEG, logits)
    m2 = jnp.max(rest, axis=1, keepdims=True)
    i2 = jnp.min(jnp.where(rest == m2, iota_f, big), axis=1, keepdims=True)
    return jnp.logical_or(sel1, iota_f == i2), i1, i2


def _k1_body(x_ref, pewt_ref, peb_ref, pdw_ref, bias_ref, base_ref, pp_ref):
    xb = x_ref[:]
    iota_p = jax.lax.broadcasted_iota(jnp.int32, (BT1, P), 1).astype(_F32)
    logits = jnp.dot(xb, pewt_ref[:], preferred_element_type=_F32) + peb_ref[:]
    mask, i1, i2 = _top2(logits, iota_p)
    pcodes = jnp.where(mask, 1.0, 0.0)
    base_ref[:] = jnp.dot(pcodes, pdw_ref[:],
                          preferred_element_type=_F32) + bias_ref[:]
    packed = i1 * 64.0 + i2
    pp_ref[:] = jnp.broadcast_to(packed, (BT1, 16)).astype(jnp.int32)


def _k1(x, pe_w, pe_b, pd_w, dec_bias):
    const = lambda *dims: pl.BlockSpec(dims, lambda i: (0,) * len(dims))
    return pl.pallas_call(
        _k1_body,
        grid=(T // BT1,),
        in_specs=[
            pl.BlockSpec((BT1, D), lambda i: (i, 0)),
            const(D, P),
            const(1, P),
            const(P, D),
            const(1, D),
        ],
        out_specs=[
            pl.BlockSpec((BT1, D), lambda i: (i, 0)),
            pl.BlockSpec((BT1, 16), lambda i: (i, 0)),
        ],
        out_shape=[
            jax.ShapeDtypeStruct((T, D), _F32),
            jax.ShapeDtypeStruct((T, 16), jnp.int32),
        ],
    )(x, pe_w.T, pe_b.reshape(1, P), pd_w, dec_bias.reshape(1, D))


def _k2a_body(pp_hbm, inv0_hbm, inv1_hbm, idx_hbm, bp_hbm, xch_hbm,
              ppv, hist_pub, histv, inv0v, inv1v, idxvf, bpv,
              runcnt, la, tots, pres, offs, blk0, nbv):
    sid = lax.axis_index("s")
    cid = lax.axis_index("c")
    tpt = T // 16
    lane = lax.iota(jnp.int32, 16)
    pltpu.sync_copy(pp_hbm.at[pl.ds(sid * tpt, tpt)], ppv)

    def zbody(p, _):
        runcnt[p] = 0
        return _

    lax.fori_loop(0, P, zbody, 0)

    def pass1(cc, _):
        for j in range(16):
            i = cc * 16 + j
            pk = ppv[i, pl.ds(0, 16)][0]
            p1 = lax.shift_right_logical(pk, 6)
            p2 = jnp.bitwise_and(pk, 63)
            r1 = runcnt[p1]
            la[2 * i] = p1 * 4096 + r1
            runcnt[p1] = r1 + 1
            r2 = runcnt[p2]
            la[2 * i + 1] = p2 * 4096 + r2
            runcnt[p2] = r2 + 1
        return _

    lax.fori_loop(0, 16, pass1, 0)

    for j in range(P // 16):
        acc = jnp.zeros((16,), jnp.int32)
        for l in range(16):
            acc = jnp.where(lane == l, runcnt[16 * j + l], acc)
        hist_pub[pl.ds(16 * j, 16)] = acc
    pltpu.sync_copy(hist_pub, xch_hbm.at[cid * 16 + sid])
    plsc.subcore_barrier()
    pltpu.sync_copy(xch_hbm.at[pl.ds(cid * 16, 16)], histv)

    for j in range(P // 16):
        tot = jnp.zeros((16,), jnp.int32)
        pre = jnp.zeros((16,), jnp.int32)
        for w2 in range(16):
            v = histv[w2, pl.ds(16 * j, 16)]
            tot = tot + v
            pre = pre + v * jnp.where(w2 < sid, 1, 0)
        for l in range(16):
            tots[16 * j + l] = tot[l]
            pres[16 * j + l] = pre[l]

    def obody(p, carry):
        t = tots[p]
        nb = lax.shift_right_logical(t + (BLK - 1), 7)
        offs[p] = carry * BLK + pres[p]
        blk0[p] = carry
        nbv[p] = nb
        return carry + nb

    totblk = lax.fori_loop(0, P, obody, 0)

    for j in range(NBLOCKS // 16):
        bpv[pl.ds(16 * j, 16)] = jnp.zeros((16,), jnp.int32)

    def bpbody(p, _):
        e = blk0[p] + nbv[p]
        for j in range(NBLOCKS // 16):
            bvec = lane + 16 * j
            sl = pl.ds(16 * j, 16)
            bpv[sl] = bpv[sl] + jnp.where(bvec >= e, 1, 0)
        return _

    lax.fori_loop(0, P, bpbody, 0)
    for j in range(NBLOCKS // 16):
        bvec = lane + 16 * j
        sl = pl.ds(16 * j, 16)
        bpv[sl] = jnp.where(bvec < totblk, bpv[sl], -1)

    @pl.when((sid == 0) & (cid == 0))
    def _bp():
        pltpu.sync_copy(bpv, bp_hbm)

    def pass2(cc, _):
        d0 = jnp.zeros((16,), jnp.int32)
        d1 = jnp.zeros((16,), jnp.int32)
        for j in range(16):
            i = cc * 16 + j
            a0 = la[2 * i]
            p0 = lax.shift_right_logical(a0, 12)
            r0 = jnp.bitwise_and(a0, 4095)
            d0 = jnp.where(lane == j, offs[p0] + r0, d0)
            a1 = la[2 * i + 1]
            p1 = lax.shift_right_logical(a1, 12)
            r1 = jnp.bitwise_and(a1, 4095)
            d1 = jnp.where(lane == j, offs[p1] + r1, d1)
        base = cc * 16
        inv0v[pl.ds(base, 16)] = d0
        inv1v[pl.ds(base, 16)] = d1
        h = lax.shift_right_logical(cc, 3)
        chunk = jnp.bitwise_and(lax.shift_right_logical(cc, 1), 3)
        colb = jnp.bitwise_and(cc, 1) * 16
        fb = h * 256 + chunk * 32 + colb
        idxvf[pl.ds(fb, 16)] = d0
        idxvf[pl.ds(fb + 128, 16)] = d1
        return _

    lax.fori_loop(0, 16, pass2, 0)

    @pl.when(cid == 0)
    def _wr():
        pltpu.sync_copy(inv0v, inv0_hbm.at[pl.ds(sid * tpt, tpt)])
        pltpu.sync_copy(inv1v, inv1_hbm.at[pl.ds(sid * tpt, tpt)])
        pltpu.sync_copy(idxvf, idx_hbm.at[pl.ds(sid * 512, 512)])


def _k2a(pp):
    mesh = plsc.VectorSubcoreMesh(core_axis_name="c", subcore_axis_name="s")
    tpt = T // 16
    f = functools.partial(
        pl.kernel,
        out_type=[
            jax.ShapeDtypeStruct((T,), jnp.int32),
            jax.ShapeDtypeStruct((T,), jnp.int32),
            jax.ShapeDtypeStruct((32 * 256,), jnp.int32),
            jax.ShapeDtypeStruct((NBLOCKS,), jnp.int32),
            jax.ShapeDtypeStruct((32, P), jnp.int32),
        ],
        mesh=mesh,
        scratch_types=[
            pltpu.VMEM((tpt, 16), jnp.int32),
            pltpu.VMEM((P,), jnp.int32),
            pltpu.VMEM((16, P), jnp.int32),
            pltpu.VMEM((tpt,), jnp.int32),
            pltpu.VMEM((tpt,), jnp.int32),
            pltpu.VMEM((512,), jnp.int32),
            pltpu.VMEM((NBLOCKS,), jnp.int32),
            pltpu.SMEM((P,), jnp.int32),
            pltpu.SMEM((2 * tpt,), jnp.int32),
            pltpu.SMEM((P,), jnp.int32),
            pltpu.SMEM((P,), jnp.int32),
            pltpu.SMEM((P,), jnp.int32),
            pltpu.SMEM((P,), jnp.int32),
            pltpu.SMEM((P,), jnp.int32),
        ],
    )(_k2a_body)
    return f(pp)


def _k2b_body(x_hbm, idx_hbm, xs_hbm, idxflat, idxv2, xra, xrb,
              sl0, sl1, ss0, ss1):
    v = lax.axis_index("s") * 2 + lax.axis_index("c")
    pltpu.sync_copy(idx_hbm.at[pl.ds(v * 256, 256)], idxflat)
    for r in range(8):
        for h in range(2):
            idxv2[r, pl.ds(h * 16, 16)] = idxflat[pl.ds(r * 32 + h * 16, 16)]
    bufs, lsems, ssems = [xra, xrb], [sl0, sl1], [ss0, ss1]

    def load(c, slot):
        return pltpu.async_copy(x_hbm.at[pl.ds(v * 128 + c * 32, 32)],
                                bufs[slot], lsems[slot])

    pend_load = load(0, 0)
    pend_scat = {0: (), 1: ()}
    for c in range(4):
        slot = c & 1
        pend_load.wait()
        if c < 3:
            for dsc in pend_scat[1 - slot]:
                dsc.wait()
            pend_load = load(c + 1, 1 - slot)
        d1 = pltpu.async_copy(bufs[slot], xs_hbm.at[idxv2.at[c]], ssems[slot])
        d2 = pltpu.async_copy(bufs[slot], xs_hbm.at[idxv2.at[4 + c]],
                              ssems[slot])
        pend_scat[slot] = (d1, d2)
    for slot in (0, 1):
        for dsc in pend_scat[slot]:
            dsc.wait()


def _k2b(x, idx):
    mesh = plsc.VectorSubcoreMesh(core_axis_name="c", subcore_axis_name="s")
    f = functools.partial(
        pl.kernel,
        out_type=[jax.ShapeDtypeStruct((NROWS, D), _F32)],
        mesh=mesh,
        scratch_types=[
            pltpu.VMEM((256,), jnp.int32),
            pltpu.VMEM((8, 32), jnp.int32),
            pltpu.VMEM((32, D), _F32),
            pltpu.VMEM((32, D), _F32),
            pltpu.SemaphoreType.DMA,
            pltpu.SemaphoreType.DMA,
            pltpu.SemaphoreType.DMA,
            pltpu.SemaphoreType.DMA,
        ],
    )(_k2b_body)
    return f(x, idx)[0]


def _k3_body(bp_ref, xs_ref, *ws_and_out):
    ws, out_ref = ws_and_out[:-1], ws_and_out[-1]
    g = pl.program_id(0)

    @pl.when(bp_ref[KPACK * g] >= 0)
    def _():
        iota_c = jax.lax.broadcasted_iota(jnp.int32, (BLK, C), 1).astype(_F32)
        for h in range(KPACK):
            dn, ce, cb, cd, up = ws[5 * h:5 * h + 5]
            xb = xs_ref[h * BLK:(h + 1) * BLK, :]
            xsub = jnp.dot(xb, dn[0], preferred_element_type=_F32)
            cl = jnp.dot(xsub, ce[0], preferred_element_type=_F32) + cb[0]
            mask, _i1, _i2 = _top2(cl, iota_c)
            codes = jnp.where(mask, 1.0, 0.0)
            csub = jnp.dot(codes, cd[0], preferred_element_type=_F32)
            out_ref[h * BLK:(h + 1) * BLK, :] = jnp.dot(
                csub, up[0], preferred_element_type=_F32)


def _k3(bp, xs, down_t, ce_t, ce_b3, cd_t, up_t):
    nsteps = NBLOCKS // KPACK
    wspecs = []
    for h in range(KPACK):
        wm = (lambda hh: lambda b, bp: (jnp.maximum(bp[KPACK * b + hh], 0),
                                        0, 0))(h)
        wspecs += [pl.BlockSpec((1, D, S), wm), pl.BlockSpec((1, S, C), wm),
                   pl.BlockSpec((1, 1, C), wm), pl.BlockSpec((1, C, S), wm),
                   pl.BlockSpec((1, S, D), wm)]
    grid_spec = pltpu.PrefetchScalarGridSpec(
        num_scalar_prefetch=1,
        grid=(nsteps,),
        in_specs=[
            pl.BlockSpec((KPACK * BLK, D),
                         lambda b, bp: (jnp.where(bp[KPACK * b] >= 0, b, 0),
                                        0)),
        ] + wspecs,
        out_specs=pl.BlockSpec(
            (KPACK * BLK, D),
            lambda b, bp: (jnp.where(bp[KPACK * b] >= 0, b, nsteps), 0)),
    )
    w = (down_t, ce_t, ce_b3, cd_t, up_t)
    return pl.pallas_call(
        _k3_body,
        grid_spec=grid_spec,
        out_shape=jax.ShapeDtypeStruct((NROWS + KPACK * BLK, D), _F32),
    )(bp, xs, *(w * KPACK))


def _k4_body(rows_hbm, base_hbm, inv0_hbm, inv1_hbm, out_hbm,
             inv0v, inv1v, b0a, b0b, b1a, b1b, bba, bbb, s0, s1):
    v = lax.axis_index("s") * 2 + lax.axis_index("c")
    t0 = v * 128
    bufs0, bufs1, bbs, sems = [b0a, b0b], [b1a, b1b], [bba, bbb], [s0, s1]
    pltpu.sync_copy(inv0_hbm.at[pl.ds(t0, 128)], inv0v)
    pltpu.sync_copy(inv1_hbm.at[pl.ds(t0, 128)], inv1v)

    def start(c, slot):
        sl = pl.ds(c * 16, 16)
        return (
            pltpu.async_copy(rows_hbm.at[inv0v.at[sl]], bufs0[slot],
                             sems[slot]),
            pltpu.async_copy(rows_hbm.at[inv1v.at[sl]], bufs1[slot],
                             sems[slot]),
            pltpu.async_copy(base_hbm.at[pl.ds(t0 + c * 16, 16)], bbs[slot],
                             sems[slot]),
        )

    pend = start(0, 0)
    for c in range(8):
        slot = c & 1
        if c < 7:
            nxt = start(c + 1, 1 - slot)
        for dsc in pend:
            dsc.wait()
        bb, buf0, buf1 = bbs[slot], bufs0[slot], bufs1[slot]

        def radd(r, _):
            for h in range(D // 16):
                sl = pl.ds(h * 16, 16)
                bb[r, sl] = bb[r, sl] + buf0[r, sl] + buf1[r, sl]
            return _

        lax.fori_loop(0, 16, radd, 0)
        pltpu.sync_copy(bb, out_hbm.at[pl.ds(t0 + c * 16, 16)])
        if c < 7:
            pend = nxt


def _k4(rows, base, inv0, inv1):
    mesh = plsc.VectorSubcoreMesh(core_axis_name="c", subcore_axis_name="s")
    f = functools.partial(
        pl.kernel,
        out_type=[jax.ShapeDtypeStruct((T, D), _F32)],
        mesh=mesh,
        scratch_types=[
            pltpu.VMEM((128,), jnp.int32),
            pltpu.VMEM((128,), jnp.int32),
            pltpu.VMEM((16, D), _F32),
            pltpu.VMEM((16, D), _F32),
            pltpu.VMEM((16, D), _F32),
            pltpu.VMEM((16, D), _F32),
            pltpu.VMEM((16, D), _F32),
            pltpu.VMEM((16, D), _F32),
            pltpu.SemaphoreType.DMA,
            pltpu.SemaphoreType.DMA,
        ],
    )(_k4_body)
    return f(rows, base, inv0, inv1)[0]


def kernel(x, pe_w, pe_b, pd_w, down_w, up_w, ce_w, ce_b, cd_w, dec_bias):
    down_t = jnp.transpose(down_w, (0, 2, 1))
    ce_t = jnp.transpose(ce_w, (0, 2, 1))
    ce_b3 = ce_b.reshape(P, 1, C)
    cd_t = jnp.transpose(cd_w, (0, 2, 1))
    up_t = jnp.transpose(up_w, (0, 2, 1))

    base, pp = _k1(x, pe_w, pe_b, pd_w, dec_bias)
    inv0, inv1, idx, bp, _ = _k2a(pp)
    xs = _k2b(x, idx)
    rows = _k3(bp, xs, down_t, ce_t, ce_b3, cd_t, up_t)
    return _k4(rows, base, inv0, inv1)
